# Initial kernel scaffold; baseline (speedup 1.0000x reference)
import jax
import jax.numpy as jnp
from jax.experimental import pallas as pl


def kernel(classic_features, edge_index, edge_type, W1_rel, W1_root, b1, W2_rel, W2_root, b2):
    raise NotImplementedError("write your pallas kernel here")


# speedup vs baseline: 10.0528x; 10.0528x over previous
import functools

import jax
import jax.numpy as jnp
from jax import lax
from jax.experimental import pallas as pl
from jax.experimental.pallas import tpu as pltpu
from jax.experimental.pallas import tpu_sc as plsc

N_NODES = 10000
N_EDGES = 320000
D = 128
N_REL = 4
NC = 2
NS = 16
NP = 10240
ROWS_PT = NP // NS
DUMMY = N_NODES + 8
EPT = N_EDGES // NS
CH = 2000
NVR = CH // 16
NCHUNK = EPT // CH
GB = 128
NR2 = (GB - 1 + CH + GB - 1) // GB + 1


def _sc_body(x_hbm, src_hbm, dst_hbm, typ_hbm, s_hbm, c_hbm,
             acc_sh, est_s, est_d, est_t, csrc, cdst, cnt_v, gbuf, sem):
    c = lax.axis_index("c")
    s = lax.axis_index("s")
    ebase = s * EPT
    rbase = s * ROWS_PT
    zv = jnp.zeros((16,), jnp.float32)
    ones = jnp.ones((16,), jnp.float32)
    iota16 = lax.iota(jnp.int32, 16)

    def _gs(j, _):
        pltpu.async_copy(x_hbm.at[csrc.at[j]], gbuf, sem).wait()
        pltpu.sync_copy(gbuf, acc_sh.at[cdst.at[j]], add=True)
        return 0

    for rpass in range(N_REL // NC):
        r = c * (N_REL // NC) + rpass

        def _zg(i, _):
            for j in range(D // 16):
                gbuf[i, pl.ds(j * 16, 16)] = zv
            return 0
        lax.fori_loop(0, GB, _zg, 0)
        for k in range(ROWS_PT // GB):
            pltpu.sync_copy(gbuf, acc_sh.at[pl.ds(rbase + k * GB, GB)])

        def _zc(i, _):
            cnt_v[pl.ds(i * 16, 16)] = zv
            return 0
        lax.fori_loop(0, NP // 16, _zc, 0)
        plsc.subcore_barrier()

        def _chunk(ci, n):
            eoff = ebase + ci * CH
            pltpu.sync_copy(src_hbm.at[pl.ds(eoff, CH)], est_s)
            pltpu.sync_copy(dst_hbm.at[pl.ds(eoff, CH)], est_d)
            pltpu.sync_copy(typ_hbm.at[pl.ds(eoff, CH)], est_t)

            def _vec(i, n):
                tv = est_t[pl.ds(i * 16, 16)]
                m = tv == r
                sv = est_s[pl.ds(i * 16, 16)]
                dv = est_d[pl.ds(i * 16, 16)]
                pos = n + plsc.cumsum(m.astype(jnp.int32)) - 1
                ph = jnp.right_shift(pos, 7)
                plo = jnp.bitwise_and(pos, GB - 1)
                plsc.store_scatter(csrc, [ph, plo], sv, mask=m)
                plsc.store_scatter(cdst, [ph, plo], dv, mask=m)
                plsc.addupdate_scatter(cnt_v, [dv], ones, mask=m)
                return n + plsc.all_reduce_population_count(m)[0]

            n = lax.fori_loop(0, NVR, _vec, n)

            nfull = jnp.right_shift(n, 7)
            lax.fori_loop(0, nfull, _gs, 0)
            for k in range(GB // 16):
                csrc[0, pl.ds(k * 16, 16)] = csrc[nfull, pl.ds(k * 16, 16)]
                cdst[0, pl.ds(k * 16, 16)] = cdst[nfull, pl.ds(k * 16, 16)]
            return jnp.bitwise_and(n, GB - 1)

        n = lax.fori_loop(0, NCHUNK, _chunk, jnp.int32(0))

        for k in range(GB // 16):
            pidx = n + k * 16 + iota16
            ph = jnp.right_shift(pidx, 7)
            plo = jnp.bitwise_and(pidx, GB - 1)
            plsc.store_scatter(csrc, [ph, plo], jnp.zeros((16,), jnp.int32))
            plsc.store_scatter(cdst, [ph, plo],
                               jnp.full((16,), DUMMY, jnp.int32))
        nch = jnp.right_shift(n + GB - 1, 7)
        lax.fori_loop(0, nch, _gs, 0)

        plsc.subcore_barrier()
        pltpu.sync_copy(acc_sh.at[pl.ds(rbase, ROWS_PT)],
                        s_hbm.at[r, pl.ds(rbase, ROWS_PT)])
        pltpu.sync_copy(cnt_v, c_hbm.at[r, s])
        plsc.subcore_barrier()


_sc_scatter = pl.kernel(
    _sc_body,
    out_type=(
        jax.ShapeDtypeStruct((N_REL, NP, D), jnp.float32),
        jax.ShapeDtypeStruct((N_REL, NS, NP), jnp.float32),
    ),
    mesh=plsc.VectorSubcoreMesh(core_axis_name="c", subcore_axis_name="s"),
    compiler_params=pltpu.CompilerParams(needs_layout_passes=False),
    scratch_types=[
        pltpu.VMEM_SHARED((NP, D), jnp.float32),
        pltpu.VMEM((CH,), jnp.int32),
        pltpu.VMEM((CH,), jnp.int32),
        pltpu.VMEM((CH,), jnp.int32),
        pltpu.VMEM((NR2, GB), jnp.int32),
        pltpu.VMEM((NR2, GB), jnp.int32),
        pltpu.VMEM((NP,), jnp.float32),
        pltpu.VMEM((GB, D), jnp.float32),
        pltpu.SemaphoreType.DMA,
    ],
)


def _tc_layer_body(s_ref, c_ref, x_ref, wrel_ref, wroot_ref, b_ref, o_ref,
                   *, relu):
    acc = jnp.dot(x_ref[...], wroot_ref[...],
                  preferred_element_type=jnp.float32) + b_ref[...]
    cnt = jnp.maximum(jnp.sum(c_ref[...], axis=1), 1.0)
    for r in range(N_REL):
        inv_r = 1.0 / cnt[r]
        mean_r = s_ref[r] * lax.broadcast_in_dim(inv_r, s_ref.shape[1:], (0,))
        acc = acc + jnp.dot(mean_r, wrel_ref[r],
                            preferred_element_type=jnp.float32)
    o_ref[...] = jnp.maximum(acc, 0.0) if relu else acc


def _tc_layer(S, C, x, W_rel, W_root, b, relu):
    k = W_rel.shape[2]
    blk = 1280
    grid = (NP // blk,)
    return pl.pallas_call(
        functools.partial(_tc_layer_body, relu=relu),
        grid=grid,
        in_specs=[
            pl.BlockSpec((N_REL, blk, D), lambda i: (0, i, 0)),
            pl.BlockSpec((N_REL, NS, blk), lambda i: (0, 0, i)),
            pl.BlockSpec((blk, D), lambda i: (i, 0)),
            pl.BlockSpec((N_REL, D, k), lambda i: (0, 0, 0)),
            pl.BlockSpec((D, k), lambda i: (0, 0)),
            pl.BlockSpec((1, k), lambda i: (0, 0)),
        ],
        out_specs=pl.BlockSpec((blk, k), lambda i: (i, 0)),
        out_shape=jax.ShapeDtypeStruct((NP, k), jnp.float32),
    )(S, C, x, W_rel, W_root, b)


def kernel(classic_features, edge_index, edge_type, W1_rel, W1_root, b1,
           W2_rel, W2_root, b2):
    src = edge_index[0].astype(jnp.int32)
    dst = edge_index[1].astype(jnp.int32)
    typ = edge_type.astype(jnp.int32)

    xp = jnp.zeros((NP, D), jnp.float32).at[:N_NODES].set(classic_features)
    S1, C1 = _sc_scatter(xp, src, dst, typ)
    h = _tc_layer(S1, C1, xp, W1_rel, W1_root, b1.reshape(1, -1), relu=True)
    S2, C2 = _sc_scatter(h, src, dst, typ)
    out = _tc_layer(S2, C2, h, W2_rel, W2_root, b2.reshape(1, -1),
                    relu=False)
    return out[:N_NODES]


# speedup vs baseline: 13.3991x; 1.3329x over previous
import functools

import jax
import jax.numpy as jnp
from jax import lax
from jax.experimental import pallas as pl
from jax.experimental.pallas import tpu as pltpu
from jax.experimental.pallas import tpu_sc as plsc

N_NODES = 10000
N_EDGES = 320000
D = 128
N_REL = 4
NC = 2
NS = 16
NP = 10240
ROWS_PT = NP // NS
DUMMY = N_NODES + 8
EPT = N_EDGES // NS
CH = 2000
NVR = CH // 16
NCHUNK = EPT // CH
GB = 64
LGB = 6
NR2 = (GB - 1 + CH + GB - 1) // GB + 1


def _sc_body(x_hbm, src_hbm, dst_hbm, typ_hbm, s_hbm, c_hbm,
             acc_sh, est_s, est_d, est_t, csrc, cdst, cnt_v, gbuf0, gbuf1,
             gsem0, gsem1):
    c = lax.axis_index("c")
    s = lax.axis_index("s")
    ebase = s * EPT
    rbase = s * ROWS_PT
    zv = jnp.zeros((16,), jnp.float32)
    ones = jnp.ones((16,), jnp.float32)
    iota16 = lax.iota(jnp.int32, 16)

    def _drain(nfull):
        @pl.when(nfull > 0)
        def _():
            pltpu.async_copy(x_hbm.at[csrc.at[0]], gbuf0, gsem0)

        def _pair(p, _):
            j0 = 2 * p
            j1 = j0 + 1

            @pl.when(j1 < nfull)
            def _():
                pltpu.async_copy(x_hbm.at[csrc.at[j1]], gbuf1, gsem1)

            pltpu.make_async_copy(x_hbm.at[csrc.at[j0]], gbuf0, gsem0).wait()
            pltpu.sync_copy(gbuf0, acc_sh.at[cdst.at[j0]], add=True)

            @pl.when(j1 < nfull)
            def _():
                @pl.when(j1 + 1 < nfull)
                def _():
                    pltpu.async_copy(x_hbm.at[csrc.at[j1 + 1]], gbuf0, gsem0)

                pltpu.make_async_copy(
                    x_hbm.at[csrc.at[j1]], gbuf1, gsem1).wait()
                pltpu.sync_copy(gbuf1, acc_sh.at[cdst.at[j1]], add=True)

            return 0

        lax.fori_loop(0, (nfull + 1) >> 1, _pair, 0)

    for rpass in range(N_REL // NC):
        r = c * (N_REL // NC) + rpass

        def _zg(i, _):
            for j in range(D // 16):
                gbuf0[i, pl.ds(j * 16, 16)] = zv
            return 0
        lax.fori_loop(0, GB, _zg, 0)
        for k in range(ROWS_PT // GB):
            pltpu.sync_copy(gbuf0, acc_sh.at[pl.ds(rbase + k * GB, GB)])

        def _zc(i, _):
            cnt_v[pl.ds(i * 16, 16)] = zv
            return 0
        lax.fori_loop(0, NP // 16, _zc, 0)
        plsc.subcore_barrier()

        def _chunk(ci, n):
            eoff = ebase + ci * CH
            pltpu.sync_copy(src_hbm.at[pl.ds(eoff, CH)], est_s)
            pltpu.sync_copy(dst_hbm.at[pl.ds(eoff, CH)], est_d)
            pltpu.sync_copy(typ_hbm.at[pl.ds(eoff, CH)], est_t)

            def _vec(i, n):
                tv = est_t[pl.ds(i * 16, 16)]
                m = tv == r
                sv = est_s[pl.ds(i * 16, 16)]
                dv = est_d[pl.ds(i * 16, 16)]
                pos = n + plsc.cumsum(m.astype(jnp.int32)) - 1
                ph = jnp.right_shift(pos, LGB)
                plo = jnp.bitwise_and(pos, GB - 1)
                plsc.store_scatter(csrc, [ph, plo], sv, mask=m)
                plsc.store_scatter(cdst, [ph, plo], dv, mask=m)
                plsc.addupdate_scatter(cnt_v, [dv], ones, mask=m)
                return n + plsc.all_reduce_population_count(m)[0]

            n = lax.fori_loop(0, NVR, _vec, n)

            nfull = jnp.right_shift(n, LGB)
            _drain(nfull)
            for k in range(GB // 16):
                csrc[0, pl.ds(k * 16, 16)] = csrc[nfull, pl.ds(k * 16, 16)]
                cdst[0, pl.ds(k * 16, 16)] = cdst[nfull, pl.ds(k * 16, 16)]
            return jnp.bitwise_and(n, GB - 1)

        n = lax.fori_loop(0, NCHUNK, _chunk, jnp.int32(0))

        for k in range(GB // 16):
            pidx = n + k * 16 + iota16
            ph = jnp.right_shift(pidx, LGB)
            plo = jnp.bitwise_and(pidx, GB - 1)
            plsc.store_scatter(csrc, [ph, plo], jnp.zeros((16,), jnp.int32))
            plsc.store_scatter(cdst, [ph, plo],
                               jnp.full((16,), DUMMY, jnp.int32))
        nch = jnp.right_shift(n + GB - 1, LGB)
        _drain(nch)

        plsc.subcore_barrier()
        pltpu.sync_copy(acc_sh.at[pl.ds(rbase, ROWS_PT)],
                        s_hbm.at[r, pl.ds(rbase, ROWS_PT)])
        pltpu.sync_copy(cnt_v, c_hbm.at[r, s])
        plsc.subcore_barrier()


_sc_scatter = pl.kernel(
    _sc_body,
    out_type=(
        jax.ShapeDtypeStruct((N_REL, NP, D), jnp.float32),
        jax.ShapeDtypeStruct((N_REL, NS, NP), jnp.float32),
    ),
    mesh=plsc.VectorSubcoreMesh(core_axis_name="c", subcore_axis_name="s"),
    compiler_params=pltpu.CompilerParams(needs_layout_passes=False),
    scratch_types=[
        pltpu.VMEM_SHARED((NP, D), jnp.float32),
        pltpu.VMEM((CH,), jnp.int32),
        pltpu.VMEM((CH,), jnp.int32),
        pltpu.VMEM((CH,), jnp.int32),
        pltpu.VMEM((NR2, GB), jnp.int32),
        pltpu.VMEM((NR2, GB), jnp.int32),
        pltpu.VMEM((NP,), jnp.float32),
        pltpu.VMEM((GB, D), jnp.float32),
        pltpu.VMEM((GB, D), jnp.float32),
        pltpu.SemaphoreType.DMA,
        pltpu.SemaphoreType.DMA,
    ],
)


def _tc_layer_body(s_ref, c_ref, x_ref, wrel_ref, wroot_ref, b_ref, o_ref,
                   *, relu):
    acc = jnp.dot(x_ref[...], wroot_ref[...],
                  preferred_element_type=jnp.float32) + b_ref[...]
    cnt = jnp.maximum(jnp.sum(c_ref[...], axis=1), 1.0)
    for r in range(N_REL):
        inv_r = 1.0 / cnt[r]
        mean_r = s_ref[r] * lax.broadcast_in_dim(inv_r, s_ref.shape[1:], (0,))
        acc = acc + jnp.dot(mean_r, wrel_ref[r],
                            preferred_element_type=jnp.float32)
    o_ref[...] = jnp.maximum(acc, 0.0) if relu else acc


def _tc_layer(S, C, x, W_rel, W_root, b, relu):
    k = W_rel.shape[2]
    blk = 1280
    grid = (NP // blk,)
    return pl.pallas_call(
        functools.partial(_tc_layer_body, relu=relu),
        grid=grid,
        in_specs=[
            pl.BlockSpec((N_REL, blk, D), lambda i: (0, i, 0)),
            pl.BlockSpec((N_REL, NS, blk), lambda i: (0, 0, i)),
            pl.BlockSpec((blk, D), lambda i: (i, 0)),
            pl.BlockSpec((N_REL, D, k), lambda i: (0, 0, 0)),
            pl.BlockSpec((D, k), lambda i: (0, 0)),
            pl.BlockSpec((1, k), lambda i: (0, 0)),
        ],
        out_specs=pl.BlockSpec((blk, k), lambda i: (i, 0)),
        out_shape=jax.ShapeDtypeStruct((NP, k), jnp.float32),
    )(S, C, x, W_rel, W_root, b)


def kernel(classic_features, edge_index, edge_type, W1_rel, W1_root, b1,
           W2_rel, W2_root, b2):
    src = edge_index[0].astype(jnp.int32)
    dst = edge_index[1].astype(jnp.int32)
    typ = edge_type.astype(jnp.int32)

    xp = jnp.zeros((NP, D), jnp.float32).at[:N_NODES].set(classic_features)
    S1, C1 = _sc_scatter(xp, src, dst, typ)
    h = _tc_layer(S1, C1, xp, W1_rel, W1_root, b1.reshape(1, -1), relu=True)
    S2, C2 = _sc_scatter(h, src, dst, typ)
    out = _tc_layer(S2, C2, h, W2_rel, W2_root, b2.reshape(1, -1),
                    relu=False)
    return out[:N_NODES]


# speedup vs baseline: 14.1259x; 1.0542x over previous
import functools

import jax
import jax.numpy as jnp
from jax import lax
from jax.experimental import pallas as pl
from jax.experimental.pallas import tpu as pltpu
from jax.experimental.pallas import tpu_sc as plsc

N_NODES = 10000
N_EDGES = 320000
D = 128
N_REL = 4
NC = 2
NS = 16
NP = 10240
ROWS_PT = NP // NS
DUMMY = N_NODES + 8
EPT = N_EDGES // NS
CH = 2000
NVR = CH // 16
NCHUNK = EPT // CH
GB = 64
LGB = 6
NR2 = 40
MAXR = (NCHUNK + 1) * NR2


def _sc_body(x_hbm, src_hbm, dst_hbm, typ_hbm, s_hbm, c_hbm,
             lsrc_hbm, ldst_hbm, ngrp_hbm,
             acc_sh, est_s, est_d, est_t, csrc, cdst, cnt_v, gbuf0, gbuf1,
             ngbuf, gsem0, gsem1, psem0, psem1):
    c = lax.axis_index("c")
    s = lax.axis_index("s")
    ebase = s * EPT
    rbase = s * ROWS_PT
    zv = jnp.zeros((16,), jnp.float32)
    ones = jnp.ones((16,), jnp.float32)
    iota16 = lax.iota(jnp.int32, 16)

    def _drain(nfull):
        @pl.when(nfull > 0)
        def _():
            pltpu.async_copy(x_hbm.at[csrc.at[0]], gbuf0, gsem0)

        def _pair(p, _):
            j0 = 2 * p
            j1 = j0 + 1

            @pl.when(j1 < nfull)
            def _():
                pltpu.async_copy(x_hbm.at[csrc.at[j1]], gbuf1, gsem1)

            pltpu.make_async_copy(x_hbm.at[csrc.at[j0]], gbuf0, gsem0).wait()
            pltpu.sync_copy(gbuf0, acc_sh.at[cdst.at[j0]], add=True)

            @pl.when(j1 < nfull)
            def _():
                @pl.when(j1 + 1 < nfull)
                def _():
                    pltpu.async_copy(x_hbm.at[csrc.at[j1 + 1]], gbuf0, gsem0)

                pltpu.make_async_copy(
                    x_hbm.at[csrc.at[j1]], gbuf1, gsem1).wait()
                pltpu.sync_copy(gbuf1, acc_sh.at[cdst.at[j1]], add=True)

            return 0

        lax.fori_loop(0, (nfull + 1) >> 1, _pair, 0)

    for rpass in range(N_REL // NC):
        r = c * (N_REL // NC) + rpass

        def _zg(i, _):
            for j in range(D // 16):
                gbuf0[i, pl.ds(j * 16, 16)] = zv
            return 0
        lax.fori_loop(0, GB, _zg, 0)
        for k in range(ROWS_PT // GB):
            pltpu.sync_copy(gbuf0, acc_sh.at[pl.ds(rbase + k * GB, GB)])

        def _zc(i, _):
            cnt_v[pl.ds(i * 16, 16)] = zv
            return 0
        lax.fori_loop(0, NP // 16, _zc, 0)
        plsc.subcore_barrier()

        def _chunk(ci, carry):
            n, ngv = carry
            eoff = ebase + ci * CH
            pltpu.sync_copy(src_hbm.at[pl.ds(eoff, CH)], est_s)
            pltpu.sync_copy(dst_hbm.at[pl.ds(eoff, CH)], est_d)
            pltpu.sync_copy(typ_hbm.at[pl.ds(eoff, CH)], est_t)

            def _vec(i, n):
                tv = est_t[pl.ds(i * 16, 16)]
                m = tv == r
                sv = est_s[pl.ds(i * 16, 16)]
                dv = est_d[pl.ds(i * 16, 16)]
                pos = n + plsc.cumsum(m.astype(jnp.int32)) - 1
                ph = jnp.right_shift(pos, LGB)
                plo = jnp.bitwise_and(pos, GB - 1)
                plsc.store_scatter(csrc, [ph, plo], sv, mask=m)
                plsc.store_scatter(cdst, [ph, plo], dv, mask=m)
                plsc.addupdate_scatter(cnt_v, [dv], ones, mask=m)
                return n + plsc.all_reduce_population_count(m)[0]

            n = lax.fori_loop(0, NVR, _vec, n)

            pltpu.async_copy(csrc, ldyn(lsrc_hbm, ci), psem0)
            pltpu.async_copy(cdst, ldyn(ldst_hbm, ci), psem1)

            nfull = jnp.right_shift(n, LGB)
            ngv = jnp.where(iota16 == ci, nfull, ngv)
            _drain(nfull)
            pltpu.make_async_copy(csrc, ldyn(lsrc_hbm, ci), psem0).wait()
            pltpu.make_async_copy(cdst, ldyn(ldst_hbm, ci), psem1).wait()
            for k in range(GB // 16):
                csrc[0, pl.ds(k * 16, 16)] = csrc[nfull, pl.ds(k * 16, 16)]
                cdst[0, pl.ds(k * 16, 16)] = cdst[nfull, pl.ds(k * 16, 16)]
            return jnp.bitwise_and(n, GB - 1), ngv

        ldyn = lambda ref, ci: ref.at[r, s, pl.ds(ci * NR2, NR2)]
        n, ngv = lax.fori_loop(
            0, NCHUNK, _chunk,
            (jnp.int32(0), jnp.zeros((16,), jnp.int32)))

        for k in range(GB // 16):
            pidx = n + k * 16 + iota16
            ph = jnp.right_shift(pidx, LGB)
            plo = jnp.bitwise_and(pidx, GB - 1)
            plsc.store_scatter(csrc, [ph, plo], jnp.zeros((16,), jnp.int32))
            plsc.store_scatter(cdst, [ph, plo],
                               jnp.full((16,), DUMMY, jnp.int32))
        nch = jnp.right_shift(n + GB - 1, LGB)
        pltpu.sync_copy(csrc.at[pl.ds(0, 8)],
                        lsrc_hbm.at[r, s, pl.ds(NCHUNK * NR2, 8)])
        pltpu.sync_copy(cdst.at[pl.ds(0, 8)],
                        ldst_hbm.at[r, s, pl.ds(NCHUNK * NR2, 8)])
        _drain(nch)

        ngv = jnp.where(iota16 == NCHUNK, nch, ngv)
        ngbuf[pl.ds(0, 16)] = ngv
        pltpu.sync_copy(ngbuf, ngrp_hbm.at[r, s])

        plsc.subcore_barrier()
        pltpu.sync_copy(acc_sh.at[pl.ds(rbase, ROWS_PT)],
                        s_hbm.at[r, pl.ds(rbase, ROWS_PT)])
        pltpu.sync_copy(cnt_v, c_hbm.at[r, s])
        plsc.subcore_barrier()


_sc_scatter = pl.kernel(
    _sc_body,
    out_type=(
        jax.ShapeDtypeStruct((N_REL, NP, D), jnp.float32),
        jax.ShapeDtypeStruct((N_REL, NS, NP), jnp.float32),
        jax.ShapeDtypeStruct((N_REL, NS, MAXR, GB), jnp.int32),
        jax.ShapeDtypeStruct((N_REL, NS, MAXR, GB), jnp.int32),
        jax.ShapeDtypeStruct((N_REL, NS, 16), jnp.int32),
    ),
    mesh=plsc.VectorSubcoreMesh(core_axis_name="c", subcore_axis_name="s"),
    compiler_params=pltpu.CompilerParams(needs_layout_passes=False),
    scratch_types=[
        pltpu.VMEM_SHARED((NP, D), jnp.float32),
        pltpu.VMEM((CH,), jnp.int32),
        pltpu.VMEM((CH,), jnp.int32),
        pltpu.VMEM((CH,), jnp.int32),
        pltpu.VMEM((NR2, GB), jnp.int32),
        pltpu.VMEM((NR2, GB), jnp.int32),
        pltpu.VMEM((NP,), jnp.float32),
        pltpu.VMEM((GB, D), jnp.float32),
        pltpu.VMEM((GB, D), jnp.float32),
        pltpu.VMEM((16,), jnp.int32),
        pltpu.SemaphoreType.DMA,
        pltpu.SemaphoreType.DMA,
        pltpu.SemaphoreType.DMA,
        pltpu.SemaphoreType.DMA,
    ],
)


def _sc_body2(x_hbm, lsrc_hbm, ldst_hbm, ngrp_hbm, s_hbm,
              acc_sh, isrc, idst, gbuf0, gbuf1, ngbuf, gsem0, gsem1):
    c = lax.axis_index("c")
    s = lax.axis_index("s")
    rbase = s * ROWS_PT
    zv = jnp.zeros((16,), jnp.float32)

    def _drain(nfull):
        @pl.when(nfull > 0)
        def _():
            pltpu.async_copy(x_hbm.at[isrc.at[0]], gbuf0, gsem0)

        def _pair(p, _):
            j0 = 2 * p
            j1 = j0 + 1

            @pl.when(j1 < nfull)
            def _():
                pltpu.async_copy(x_hbm.at[isrc.at[j1]], gbuf1, gsem1)

            pltpu.make_async_copy(x_hbm.at[isrc.at[j0]], gbuf0, gsem0).wait()
            pltpu.sync_copy(gbuf0, acc_sh.at[idst.at[j0]], add=True)

            @pl.when(j1 < nfull)
            def _():
                @pl.when(j1 + 1 < nfull)
                def _():
                    pltpu.async_copy(x_hbm.at[isrc.at[j1 + 1]], gbuf0, gsem0)

                pltpu.make_async_copy(
                    x_hbm.at[isrc.at[j1]], gbuf1, gsem1).wait()
                pltpu.sync_copy(gbuf1, acc_sh.at[idst.at[j1]], add=True)

            return 0

        lax.fori_loop(0, (nfull + 1) >> 1, _pair, 0)

    for rpass in range(N_REL // NC):
        r = c * (N_REL // NC) + rpass

        def _zg(i, _):
            for j in range(D // 16):
                gbuf0[i, pl.ds(j * 16, 16)] = zv
            return 0
        lax.fori_loop(0, GB, _zg, 0)
        for k in range(ROWS_PT // GB):
            pltpu.sync_copy(gbuf0, acc_sh.at[pl.ds(rbase + k * GB, GB)])
        plsc.subcore_barrier()

        pltpu.sync_copy(ngrp_hbm.at[r, s], ngbuf)
        ngv = ngbuf[pl.ds(0, 16)]
        iota16 = lax.iota(jnp.int32, 16)

        for ci in range(NCHUNK + 1):
            cnt = jnp.sum(jnp.where(iota16 == ci, ngv, 0))
            pltpu.sync_copy(lsrc_hbm.at[r, s, pl.ds(ci * NR2, NR2)], isrc)
            pltpu.sync_copy(ldst_hbm.at[r, s, pl.ds(ci * NR2, NR2)], idst)
            _drain(cnt)

        plsc.subcore_barrier()
        pltpu.sync_copy(acc_sh.at[pl.ds(rbase, ROWS_PT)],
                        s_hbm.at[r, pl.ds(rbase, ROWS_PT)])
        plsc.subcore_barrier()


_sc_scatter2 = pl.kernel(
    _sc_body2,
    out_type=jax.ShapeDtypeStruct((N_REL, NP, D), jnp.float32),
    mesh=plsc.VectorSubcoreMesh(core_axis_name="c", subcore_axis_name="s"),
    compiler_params=pltpu.CompilerParams(needs_layout_passes=False),
    scratch_types=[
        pltpu.VMEM_SHARED((NP, D), jnp.float32),
        pltpu.VMEM((NR2, GB), jnp.int32),
        pltpu.VMEM((NR2, GB), jnp.int32),
        pltpu.VMEM((GB, D), jnp.float32),
        pltpu.VMEM((GB, D), jnp.float32),
        pltpu.VMEM((16,), jnp.int32),
        pltpu.SemaphoreType.DMA,
        pltpu.SemaphoreType.DMA,
    ],
)


def _tc_layer_body(s_ref, c_ref, x_ref, wrel_ref, wroot_ref, b_ref, o_ref,
                   *, relu):
    acc = jnp.dot(x_ref[...], wroot_ref[...],
                  preferred_element_type=jnp.float32) + b_ref[...]
    cnt = jnp.maximum(jnp.sum(c_ref[...], axis=1), 1.0)
    for r in range(N_REL):
        inv_r = 1.0 / cnt[r]
        mean_r = s_ref[r] * lax.broadcast_in_dim(inv_r, s_ref.shape[1:], (0,))
        acc = acc + jnp.dot(mean_r, wrel_ref[r],
                            preferred_element_type=jnp.float32)
    o_ref[...] = jnp.maximum(acc, 0.0) if relu else acc


def _tc_layer(S, C, x, W_rel, W_root, b, relu):
    k = W_rel.shape[2]
    blk = 1280
    grid = (NP // blk,)
    return pl.pallas_call(
        functools.partial(_tc_layer_body, relu=relu),
        grid=grid,
        in_specs=[
            pl.BlockSpec((N_REL, blk, D), lambda i: (0, i, 0)),
            pl.BlockSpec((N_REL, NS, blk), lambda i: (0, 0, i)),
            pl.BlockSpec((blk, D), lambda i: (i, 0)),
            pl.BlockSpec((N_REL, D, k), lambda i: (0, 0, 0)),
            pl.BlockSpec((D, k), lambda i: (0, 0)),
            pl.BlockSpec((1, k), lambda i: (0, 0)),
        ],
        out_specs=pl.BlockSpec((blk, k), lambda i: (i, 0)),
        out_shape=jax.ShapeDtypeStruct((NP, k), jnp.float32),
    )(S, C, x, W_rel, W_root, b)


def kernel(classic_features, edge_index, edge_type, W1_rel, W1_root, b1,
           W2_rel, W2_root, b2):
    src = edge_index[0].astype(jnp.int32)
    dst = edge_index[1].astype(jnp.int32)
    typ = edge_type.astype(jnp.int32)

    xp = jnp.zeros((NP, D), jnp.float32).at[:N_NODES].set(classic_features)
    S1, C1, LS, LD, NG = _sc_scatter(xp, src, dst, typ)
    h = _tc_layer(S1, C1, xp, W1_rel, W1_root, b1.reshape(1, -1), relu=True)
    S2 = _sc_scatter2(h, LS, LD, NG)
    out = _tc_layer(S2, C1, h, W2_rel, W2_root, b2.reshape(1, -1),
                    relu=False)
    return out[:N_NODES]


# speedup vs baseline: 15.4870x; 1.0964x over previous
import functools

import jax
import jax.numpy as jnp
from jax import lax
from jax.experimental import pallas as pl
from jax.experimental.pallas import tpu as pltpu
from jax.experimental.pallas import tpu_sc as plsc

N_NODES = 10000
N_EDGES = 320000
D = 128
N_REL = 4
NC = 2
NS = 16
NP = 10240
ROWS_PT = NP // NS
DUMMY = N_NODES + 8
EPT = N_EDGES // NS
CH = 2000
NVR = CH // 16
NCHUNK = EPT // CH
GB = 64
LGB = 6
NR2 = 40
MAXR = (NCHUNK + 1) * NR2


def _make_drain(x_hbm, acc_sh, isrc, idst, bufs, gsems, ssems):
    def start_g(j, b):
        pltpu.async_copy(x_hbm.at[isrc.at[j]], bufs[b], gsems[b])

    def wait_g(b):
        pltpu.make_async_copy(x_hbm.at[isrc.at[0]], bufs[b], gsems[b]).wait()

    def start_s(j, b):
        pltpu.async_copy(bufs[b], acc_sh.at[idst.at[j]], ssems[b], add=True)

    def wait_s(b):
        pltpu.make_async_copy(bufs[b], acc_sh.at[idst.at[0]], ssems[b]).wait()

    def drain(nfull):
        @pl.when(nfull > 0)
        def _():
            start_g(0, 0)

        @pl.when(nfull > 1)
        def _():
            start_g(1, 1)

        def _step(p, _):
            for k in range(3):
                j = 3 * p + k

                @pl.when(j < nfull)
                def _(j=j, k=k):
                    wait_g(k)
                    start_s(j, k)

                    @pl.when(j >= 1)
                    def _():
                        wait_s((k + 2) % 3)

                    @pl.when(j + 2 < nfull)
                    def _():
                        start_g(j + 2, (k + 2) % 3)

            return 0

        lax.fori_loop(0, (nfull + 2) // 3, _step, 0)
        rem = lax.rem(nfull - 1, 3)
        for k in range(3):
            @pl.when(jnp.logical_and(nfull > 0, rem == k))
            def _(k=k):
                wait_s(k)

    return drain


def _sc_body(x_hbm, edges_hbm, s_hbm, c_hbm,
             lsrc_hbm, ldst_hbm, ngrp_hbm,
             acc_sh, est, csrc, cdst, cnt_v,
             gbuf0, gbuf1, gbuf2, ngbuf,
             gsem0, gsem1, gsem2, ssem0, ssem1, ssem2, psem0, psem1):
    c = lax.axis_index("c")
    s = lax.axis_index("s")
    ebase = s * EPT
    rbase = s * ROWS_PT
    zv = jnp.zeros((16,), jnp.float32)
    ones = jnp.ones((16,), jnp.float32)
    iota16 = lax.iota(jnp.int32, 16)

    _drain = _make_drain(x_hbm, acc_sh, csrc, cdst,
                         (gbuf0, gbuf1, gbuf2),
                         (gsem0, gsem1, gsem2), (ssem0, ssem1, ssem2))

    for rpass in range(N_REL // NC):
        r = c * (N_REL // NC) + rpass

        def _zg(i, _):
            for j in range(D // 16):
                gbuf0[i, pl.ds(j * 16, 16)] = zv
            return 0
        lax.fori_loop(0, GB, _zg, 0)
        for k in range(ROWS_PT // GB):
            pltpu.sync_copy(gbuf0, acc_sh.at[pl.ds(rbase + k * GB, GB)])

        def _zc(i, _):
            cnt_v[pl.ds(i * 16, 16)] = zv
            return 0
        lax.fori_loop(0, NP // 16, _zc, 0)
        plsc.subcore_barrier()

        def _chunk(ci, carry):
            n, ngv = carry
            eoff = ebase + ci * CH
            pltpu.sync_copy(edges_hbm.at[pl.ds(eoff, CH)], est)

            def _vec(i, n):
                w = est[pl.ds(i * 16, 16)]
                tv = jnp.right_shift(w, 28)
                m = tv == r
                sv = jnp.bitwise_and(w, 16383)
                dv = jnp.bitwise_and(jnp.right_shift(w, 14), 16383)
                pos = n + plsc.cumsum(m.astype(jnp.int32)) - 1
                ph = jnp.right_shift(pos, LGB)
                plo = jnp.bitwise_and(pos, GB - 1)
                plsc.store_scatter(csrc, [ph, plo], sv, mask=m)
                plsc.store_scatter(cdst, [ph, plo], dv, mask=m)
                plsc.addupdate_scatter(cnt_v, [dv], ones, mask=m)
                return n + plsc.all_reduce_population_count(m)[0]

            n = lax.fori_loop(0, NVR, _vec, n)

            pltpu.async_copy(csrc, ldyn(lsrc_hbm, ci), psem0)
            pltpu.async_copy(cdst, ldyn(ldst_hbm, ci), psem1)

            nfull = jnp.right_shift(n, LGB)
            ngv = jnp.where(iota16 == ci, nfull, ngv)
            _drain(nfull)
            pltpu.make_async_copy(csrc, ldyn(lsrc_hbm, ci), psem0).wait()
            pltpu.make_async_copy(cdst, ldyn(ldst_hbm, ci), psem1).wait()
            for k in range(GB // 16):
                csrc[0, pl.ds(k * 16, 16)] = csrc[nfull, pl.ds(k * 16, 16)]
                cdst[0, pl.ds(k * 16, 16)] = cdst[nfull, pl.ds(k * 16, 16)]
            return jnp.bitwise_and(n, GB - 1), ngv

        ldyn = lambda ref, ci: ref.at[r, s, pl.ds(ci * NR2, NR2)]
        n, ngv = lax.fori_loop(
            0, NCHUNK, _chunk,
            (jnp.int32(0), jnp.zeros((16,), jnp.int32)))

        for k in range(GB // 16):
            pidx = n + k * 16 + iota16
            ph = jnp.right_shift(pidx, LGB)
            plo = jnp.bitwise_and(pidx, GB - 1)
            plsc.store_scatter(csrc, [ph, plo], jnp.zeros((16,), jnp.int32))
            plsc.store_scatter(cdst, [ph, plo],
                               jnp.full((16,), DUMMY, jnp.int32))
        nch = jnp.right_shift(n + GB - 1, LGB)
        pltpu.sync_copy(csrc.at[pl.ds(0, 8)],
                        lsrc_hbm.at[r, s, pl.ds(NCHUNK * NR2, 8)])
        pltpu.sync_copy(cdst.at[pl.ds(0, 8)],
                        ldst_hbm.at[r, s, pl.ds(NCHUNK * NR2, 8)])
        _drain(nch)

        ngv = jnp.where(iota16 == NCHUNK, nch, ngv)
        ngbuf[pl.ds(0, 16)] = ngv
        pltpu.sync_copy(ngbuf, ngrp_hbm.at[r, s])

        plsc.subcore_barrier()
        pltpu.sync_copy(acc_sh.at[pl.ds(rbase, ROWS_PT)],
                        s_hbm.at[r, pl.ds(rbase, ROWS_PT)])
        pltpu.sync_copy(cnt_v, c_hbm.at[r, s])
        plsc.subcore_barrier()


_sc_scatter = pl.kernel(
    _sc_body,
    out_type=(
        jax.ShapeDtypeStruct((N_REL, NP, D), jnp.float32),
        jax.ShapeDtypeStruct((N_REL, NS, NP), jnp.float32),
        jax.ShapeDtypeStruct((N_REL, NS, MAXR, GB), jnp.int32),
        jax.ShapeDtypeStruct((N_REL, NS, MAXR, GB), jnp.int32),
        jax.ShapeDtypeStruct((N_REL, NS, 16), jnp.int32),
    ),
    mesh=plsc.VectorSubcoreMesh(core_axis_name="c", subcore_axis_name="s"),
    compiler_params=pltpu.CompilerParams(needs_layout_passes=False),
    scratch_types=[
        pltpu.VMEM_SHARED((NP, D), jnp.float32),
        pltpu.VMEM((CH,), jnp.int32),
        pltpu.VMEM((NR2, GB), jnp.int32),
        pltpu.VMEM((NR2, GB), jnp.int32),
        pltpu.VMEM((NP,), jnp.float32),
        pltpu.VMEM((GB, D), jnp.float32),
        pltpu.VMEM((GB, D), jnp.float32),
        pltpu.VMEM((GB, D), jnp.float32),
        pltpu.VMEM((16,), jnp.int32),
        pltpu.SemaphoreType.DMA,
        pltpu.SemaphoreType.DMA,
        pltpu.SemaphoreType.DMA,
        pltpu.SemaphoreType.DMA,
        pltpu.SemaphoreType.DMA,
        pltpu.SemaphoreType.DMA,
        pltpu.SemaphoreType.DMA,
        pltpu.SemaphoreType.DMA,
    ],
)


def _sc_body2(x_hbm, lsrc_hbm, ldst_hbm, ngrp_hbm, s_hbm,
              acc_sh, isrc, idst, gbuf0, gbuf1, gbuf2, ngbuf,
              gsem0, gsem1, gsem2, ssem0, ssem1, ssem2):
    c = lax.axis_index("c")
    s = lax.axis_index("s")
    rbase = s * ROWS_PT
    zv = jnp.zeros((16,), jnp.float32)

    _drain = _make_drain(x_hbm, acc_sh, isrc, idst,
                         (gbuf0, gbuf1, gbuf2),
                         (gsem0, gsem1, gsem2), (ssem0, ssem1, ssem2))

    for rpass in range(N_REL // NC):
        r = c * (N_REL // NC) + rpass

        def _zg(i, _):
            for j in range(D // 16):
                gbuf0[i, pl.ds(j * 16, 16)] = zv
            return 0
        lax.fori_loop(0, GB, _zg, 0)
        for k in range(ROWS_PT // GB):
            pltpu.sync_copy(gbuf0, acc_sh.at[pl.ds(rbase + k * GB, GB)])
        plsc.subcore_barrier()

        pltpu.sync_copy(ngrp_hbm.at[r, s], ngbuf)
        ngv = ngbuf[pl.ds(0, 16)]
        iota16 = lax.iota(jnp.int32, 16)

        for ci in range(NCHUNK + 1):
            cnt = jnp.sum(jnp.where(iota16 == ci, ngv, 0))
            pltpu.sync_copy(lsrc_hbm.at[r, s, pl.ds(ci * NR2, NR2)], isrc)
            pltpu.sync_copy(ldst_hbm.at[r, s, pl.ds(ci * NR2, NR2)], idst)
            _drain(cnt)

        plsc.subcore_barrier()
        pltpu.sync_copy(acc_sh.at[pl.ds(rbase, ROWS_PT)],
                        s_hbm.at[r, pl.ds(rbase, ROWS_PT)])
        plsc.subcore_barrier()


_sc_scatter2 = pl.kernel(
    _sc_body2,
    out_type=jax.ShapeDtypeStruct((N_REL, NP, D), jnp.float32),
    mesh=plsc.VectorSubcoreMesh(core_axis_name="c", subcore_axis_name="s"),
    compiler_params=pltpu.CompilerParams(needs_layout_passes=False),
    scratch_types=[
        pltpu.VMEM_SHARED((NP, D), jnp.float32),
        pltpu.VMEM((NR2, GB), jnp.int32),
        pltpu.VMEM((NR2, GB), jnp.int32),
        pltpu.VMEM((GB, D), jnp.float32),
        pltpu.VMEM((GB, D), jnp.float32),
        pltpu.VMEM((GB, D), jnp.float32),
        pltpu.VMEM((16,), jnp.int32),
        pltpu.SemaphoreType.DMA,
        pltpu.SemaphoreType.DMA,
        pltpu.SemaphoreType.DMA,
        pltpu.SemaphoreType.DMA,
        pltpu.SemaphoreType.DMA,
        pltpu.SemaphoreType.DMA,
    ],
)


def _tc_layer_body(s_ref, c_ref, x_ref, wrel_ref, wroot_ref, b_ref, o_ref,
                   *, relu):
    acc = jnp.dot(x_ref[...], wroot_ref[...],
                  preferred_element_type=jnp.float32) + b_ref[...]
    cnt = jnp.maximum(jnp.sum(c_ref[...], axis=1), 1.0)
    for r in range(N_REL):
        inv_r = 1.0 / cnt[r]
        mean_r = s_ref[r] * lax.broadcast_in_dim(inv_r, s_ref.shape[1:], (0,))
        acc = acc + jnp.dot(mean_r, wrel_ref[r],
                            preferred_element_type=jnp.float32)
    o_ref[...] = jnp.maximum(acc, 0.0) if relu else acc


def _tc_layer(S, C, x, W_rel, W_root, b, relu):
    k = W_rel.shape[2]
    blk = 1280
    grid = (NP // blk,)
    return pl.pallas_call(
        functools.partial(_tc_layer_body, relu=relu),
        grid=grid,
        in_specs=[
            pl.BlockSpec((N_REL, blk, D), lambda i: (0, i, 0)),
            pl.BlockSpec((N_REL, NS, blk), lambda i: (0, 0, i)),
            pl.BlockSpec((blk, D), lambda i: (i, 0)),
            pl.BlockSpec((N_REL, D, k), lambda i: (0, 0, 0)),
            pl.BlockSpec((D, k), lambda i: (0, 0)),
            pl.BlockSpec((1, k), lambda i: (0, 0)),
        ],
        out_specs=pl.BlockSpec((blk, k), lambda i: (i, 0)),
        out_shape=jax.ShapeDtypeStruct((NP, k), jnp.float32),
    )(S, C, x, W_rel, W_root, b)


def kernel(classic_features, edge_index, edge_type, W1_rel, W1_root, b1,
           W2_rel, W2_root, b2):
    src = edge_index[0].astype(jnp.int32)
    dst = edge_index[1].astype(jnp.int32)
    typ = edge_type.astype(jnp.int32)
    edges = src | (dst << 14) | (typ << 28)

    xp = jnp.zeros((NP, D), jnp.float32).at[:N_NODES].set(classic_features)
    S1, C1, LS, LD, NG = _sc_scatter(xp, edges)
    h = _tc_layer(S1, C1, xp, W1_rel, W1_root, b1.reshape(1, -1), relu=True)
    S2 = _sc_scatter2(h, LS, LD, NG)
    out = _tc_layer(S2, C1, h, W2_rel, W2_root, b2.reshape(1, -1),
                    relu=False)
    return out[:N_NODES]


# speedup vs baseline: 16.7608x; 1.0822x over previous
import functools

import jax
import jax.numpy as jnp
from jax import lax
from jax.experimental import pallas as pl
from jax.experimental.pallas import tpu as pltpu
from jax.experimental.pallas import tpu_sc as plsc

N_NODES = 10000
N_EDGES = 320000
D = 128
N_REL = 4
NC = 2
NS = 16
NP = 10240
ROWS_PT = NP // NS
DUMMY = N_NODES + 8
EPT = N_EDGES // NS
CH = 2000
NVR = CH // 16
NCHUNK = EPT // CH
GB = 64
LGB = 6
NR2 = 40
MAXR = 360


def _make_drain(x_hbm, acc_sh, isrc, idst, bufs, gsems, ssems):
    def start_g(base, j, b):
        pltpu.async_copy(x_hbm.at[isrc.at[base + j]], bufs[b], gsems[b])

    def wait_g(b):
        pltpu.make_async_copy(x_hbm.at[isrc.at[0]], bufs[b], gsems[b]).wait()

    def start_s(base, j, b):
        pltpu.async_copy(bufs[b], acc_sh.at[idst.at[base + j]], ssems[b],
                         add=True)

    def wait_s(b):
        pltpu.make_async_copy(bufs[b], acc_sh.at[idst.at[0]], ssems[b]).wait()

    def drain(base, cnt):
        @pl.when(cnt > 0)
        def _():
            start_g(base, 0, 0)

        @pl.when(cnt > 1)
        def _():
            start_g(base, 1, 1)

        def _step(p, _):
            for k in range(3):
                j = 3 * p + k

                @pl.when(j < cnt)
                def _(j=j, k=k):
                    wait_g(k)
                    start_s(base, j, k)

                    @pl.when(j >= 1)
                    def _():
                        wait_s((k + 2) % 3)

                    @pl.when(j + 2 < cnt)
                    def _():
                        start_g(base, j + 2, (k + 2) % 3)

            return 0

        lax.fori_loop(0, (cnt + 2) // 3, _step, 0)
        rem = lax.rem(cnt - 1, 3)
        for k in range(3):
            @pl.when(jnp.logical_and(cnt > 0, rem == k))
            def _(k=k):
                wait_s(k)

    return drain


def _sc_body(x_hbm, edges_hbm, s_hbm, c_hbm,
             lsrc_hbm, ldst_hbm, ngrp_hbm,
             acc_sh, est, csrc, cdst, cnt_v,
             gbuf0, gbuf1, gbuf2, ngbuf,
             gsem0, gsem1, gsem2, ssem0, ssem1, ssem2, psem0, psem1):
    c = lax.axis_index("c")
    s = lax.axis_index("s")
    ebase = s * EPT
    rbase = s * ROWS_PT
    zv = jnp.zeros((16,), jnp.float32)
    ones = jnp.ones((16,), jnp.float32)
    iota16 = lax.iota(jnp.int32, 16)

    _drain = _make_drain(x_hbm, acc_sh, csrc, cdst,
                         (gbuf0, gbuf1, gbuf2),
                         (gsem0, gsem1, gsem2), (ssem0, ssem1, ssem2))

    for rpass in range(N_REL // NC):
        r = c * (N_REL // NC) + rpass

        def _zg(i, _):
            for j in range(D // 16):
                gbuf0[i, pl.ds(j * 16, 16)] = zv
            return 0
        lax.fori_loop(0, GB, _zg, 0)
        for k in range(ROWS_PT // GB):
            pltpu.sync_copy(gbuf0, acc_sh.at[pl.ds(rbase + k * GB, GB)])

        def _zc(i, _):
            cnt_v[pl.ds(i * 16, 16)] = zv
            return 0
        lax.fori_loop(0, NP // 16, _zc, 0)
        plsc.subcore_barrier()

        def _chunk(ci, carry):
            n, pbase = carry
            prow = jnp.right_shift(n, LGB)
            eoff = ebase + ci * CH
            pltpu.sync_copy(edges_hbm.at[pl.ds(eoff, CH)], est)

            def _vec(i, n):
                w = est[pl.ds(i * 16, 16)]
                tv = jnp.right_shift(w, 28)
                m = tv == r
                sv = jnp.bitwise_and(w, 16383)
                dv = jnp.bitwise_and(jnp.right_shift(w, 14), 16383)
                pos = n + plsc.cumsum(m.astype(jnp.int32)) - 1
                ph = jnp.right_shift(pos, LGB)
                plo = jnp.bitwise_and(pos, GB - 1)
                plsc.store_scatter(csrc, [ph, plo], sv, mask=m)
                plsc.store_scatter(cdst, [ph, plo], dv, mask=m)
                plsc.addupdate_scatter(cnt_v, [dv], ones, mask=m)
                return n + plsc.all_reduce_population_count(m)[0]

            n = lax.fori_loop(0, NVR, _vec, n)

            pb8 = pl.multiple_of(pbase, 8)
            pltpu.async_copy(csrc, lsrc_hbm.at[r, s, pl.ds(pb8, NR2)], psem0)
            pltpu.async_copy(cdst, ldst_hbm.at[r, s, pl.ds(pb8, NR2)], psem1)

            nfull = jnp.right_shift(n, LGB)
            _drain(prow, nfull - prow)
            pltpu.make_async_copy(
                csrc, lsrc_hbm.at[r, s, pl.ds(pb8, NR2)], psem0).wait()
            pltpu.make_async_copy(
                cdst, ldst_hbm.at[r, s, pl.ds(pb8, NR2)], psem1).wait()

            pfull8 = jnp.bitwise_and(nfull, ~7)
            for k in range(8):
                for q in range(GB // 16):
                    csrc[k, pl.ds(q * 16, 16)] = (
                        csrc[pfull8 + k, pl.ds(q * 16, 16)])
                    cdst[k, pl.ds(q * 16, 16)] = (
                        cdst[pfull8 + k, pl.ds(q * 16, 16)])
            return n - pfull8 * GB, pbase + pfull8

        n, pbase = lax.fori_loop(0, NCHUNK, _chunk,
                                 (jnp.int32(0), jnp.int32(0)))

        prow = jnp.right_shift(n, LGB)
        for k in range(GB // 16):
            pidx = n + k * 16 + iota16
            ph = jnp.right_shift(pidx, LGB)
            plo = jnp.bitwise_and(pidx, GB - 1)
            plsc.store_scatter(csrc, [ph, plo], jnp.zeros((16,), jnp.int32))
            plsc.store_scatter(cdst, [ph, plo],
                               jnp.full((16,), DUMMY, jnp.int32))
        rows_f = jnp.right_shift(n + GB - 1, LGB)
        pb8 = pl.multiple_of(pbase, 8)
        pltpu.async_copy(csrc.at[pl.ds(0, 16)],
                         lsrc_hbm.at[r, s, pl.ds(pb8, 16)], psem0)
        pltpu.async_copy(cdst.at[pl.ds(0, 16)],
                         ldst_hbm.at[r, s, pl.ds(pb8, 16)], psem1)
        _drain(prow, rows_f - prow)
        pltpu.make_async_copy(
            csrc.at[pl.ds(0, 16)], lsrc_hbm.at[r, s, pl.ds(pb8, 16)],
            psem0).wait()
        pltpu.make_async_copy(
            cdst.at[pl.ds(0, 16)], ldst_hbm.at[r, s, pl.ds(pb8, 16)],
            psem1).wait()

        ngbuf[pl.ds(0, 16)] = jnp.full((16,), 1, jnp.int32) * (pbase + rows_f)
        pltpu.sync_copy(ngbuf, ngrp_hbm.at[r, s])

        plsc.subcore_barrier()
        pltpu.sync_copy(acc_sh.at[pl.ds(rbase, ROWS_PT)],
                        s_hbm.at[r, pl.ds(rbase, ROWS_PT)])
        pltpu.sync_copy(cnt_v, c_hbm.at[r, s])
        plsc.subcore_barrier()


_sc_scatter = pl.kernel(
    _sc_body,
    out_type=(
        jax.ShapeDtypeStruct((N_REL, NP, D), jnp.float32),
        jax.ShapeDtypeStruct((N_REL, NS, NP), jnp.float32),
        jax.ShapeDtypeStruct((N_REL, NS, MAXR, GB), jnp.int32),
        jax.ShapeDtypeStruct((N_REL, NS, MAXR, GB), jnp.int32),
        jax.ShapeDtypeStruct((N_REL, NS, 16), jnp.int32),
    ),
    mesh=plsc.VectorSubcoreMesh(core_axis_name="c", subcore_axis_name="s"),
    compiler_params=pltpu.CompilerParams(needs_layout_passes=False),
    scratch_types=[
        pltpu.VMEM_SHARED((NP, D), jnp.float32),
        pltpu.VMEM((CH,), jnp.int32),
        pltpu.VMEM((NR2, GB), jnp.int32),
        pltpu.VMEM((NR2, GB), jnp.int32),
        pltpu.VMEM((NP,), jnp.float32),
        pltpu.VMEM((GB, D), jnp.float32),
        pltpu.VMEM((GB, D), jnp.float32),
        pltpu.VMEM((GB, D), jnp.float32),
        pltpu.VMEM((16,), jnp.int32),
        pltpu.SemaphoreType.DMA,
        pltpu.SemaphoreType.DMA,
        pltpu.SemaphoreType.DMA,
        pltpu.SemaphoreType.DMA,
        pltpu.SemaphoreType.DMA,
        pltpu.SemaphoreType.DMA,
        pltpu.SemaphoreType.DMA,
        pltpu.SemaphoreType.DMA,
    ],
)


def _sc_body2(x_hbm, lsrc_hbm, ldst_hbm, ngrp_hbm, s_hbm,
              acc_sh, isrc_a, idst_a, isrc_b, idst_b,
              gbuf0, gbuf1, gbuf2, ngbuf,
              gsem0, gsem1, gsem2, ssem0, ssem1, ssem2,
              psa0, psa1, psb0, psb1):
    c = lax.axis_index("c")
    s = lax.axis_index("s")
    rbase = s * ROWS_PT
    zv = jnp.zeros((16,), jnp.float32)

    bufs = (gbuf0, gbuf1, gbuf2)
    gsems = (gsem0, gsem1, gsem2)
    ssems = (ssem0, ssem1, ssem2)
    _drain_a = _make_drain(x_hbm, acc_sh, isrc_a, idst_a, bufs, gsems, ssems)
    _drain_b = _make_drain(x_hbm, acc_sh, isrc_b, idst_b, bufs, gsems, ssems)

    def _stage(blk, isrc, idst, r, sem0, sem1):
        pltpu.async_copy(lsrc_hbm.at[r, s, pl.ds(blk * NR2, NR2)], isrc,
                         sem0)
        pltpu.async_copy(ldst_hbm.at[r, s, pl.ds(blk * NR2, NR2)], idst,
                         sem1)

    def _stage_wait(blk, isrc, idst, r, sem0, sem1):
        pltpu.make_async_copy(lsrc_hbm.at[r, s, pl.ds(blk * NR2, NR2)],
                              isrc, sem0).wait()
        pltpu.make_async_copy(ldst_hbm.at[r, s, pl.ds(blk * NR2, NR2)],
                              idst, sem1).wait()

    for rpass in range(N_REL // NC):
        r = c * (N_REL // NC) + rpass

        def _zg(i, _):
            for j in range(D // 16):
                gbuf0[i, pl.ds(j * 16, 16)] = zv
            return 0
        lax.fori_loop(0, GB, _zg, 0)
        for k in range(ROWS_PT // GB):
            pltpu.sync_copy(gbuf0, acc_sh.at[pl.ds(rbase + k * GB, GB)])
        plsc.subcore_barrier()

        pltpu.sync_copy(ngrp_hbm.at[r, s], ngbuf)
        ng = ngbuf[pl.ds(0, 16)][0]
        nblk = (ng + NR2 - 1) // NR2

        @pl.when(nblk > 0)
        def _():
            _stage(0, isrc_a, idst_a, r, psa0, psa1)
            _stage_wait(0, isrc_a, idst_a, r, psa0, psa1)

        def _pstep(p, _):
            k0 = 2 * p
            k1 = k0 + 1

            @pl.when(k1 < nblk)
            def _():
                _stage(k1, isrc_b, idst_b, r, psb0, psb1)

            @pl.when(k0 >= 1)
            def _():
                _stage_wait(k0, isrc_a, idst_a, r, psa0, psa1)

            _drain_a(0, jnp.minimum(ng - k0 * NR2, NR2))

            @pl.when(k1 < nblk)
            def _():
                @pl.when(k1 + 1 < nblk)
                def _():
                    _stage(k1 + 1, isrc_a, idst_a, r, psa0, psa1)

                _stage_wait(k1, isrc_b, idst_b, r, psb0, psb1)
                _drain_b(0, jnp.minimum(ng - k1 * NR2, NR2))

            return 0

        lax.fori_loop(0, (nblk + 1) >> 1, _pstep, 0)

        plsc.subcore_barrier()
        pltpu.sync_copy(acc_sh.at[pl.ds(rbase, ROWS_PT)],
                        s_hbm.at[r, pl.ds(rbase, ROWS_PT)])
        plsc.subcore_barrier()


_sc_scatter2 = pl.kernel(
    _sc_body2,
    out_type=jax.ShapeDtypeStruct((N_REL, NP, D), jnp.float32),
    mesh=plsc.VectorSubcoreMesh(core_axis_name="c", subcore_axis_name="s"),
    compiler_params=pltpu.CompilerParams(needs_layout_passes=False),
    scratch_types=[
        pltpu.VMEM_SHARED((NP, D), jnp.float32),
        pltpu.VMEM((NR2, GB), jnp.int32),
        pltpu.VMEM((NR2, GB), jnp.int32),
        pltpu.VMEM((NR2, GB), jnp.int32),
        pltpu.VMEM((NR2, GB), jnp.int32),
        pltpu.VMEM((GB, D), jnp.float32),
        pltpu.VMEM((GB, D), jnp.float32),
        pltpu.VMEM((GB, D), jnp.float32),
        pltpu.VMEM((16,), jnp.int32),
        pltpu.SemaphoreType.DMA,
        pltpu.SemaphoreType.DMA,
        pltpu.SemaphoreType.DMA,
        pltpu.SemaphoreType.DMA,
        pltpu.SemaphoreType.DMA,
        pltpu.SemaphoreType.DMA,
        pltpu.SemaphoreType.DMA,
        pltpu.SemaphoreType.DMA,
        pltpu.SemaphoreType.DMA,
        pltpu.SemaphoreType.DMA,
    ],
)


def _tc_layer_body(s_ref, c_ref, x_ref, wrel_ref, wroot_ref, b_ref, o_ref,
                   *, relu):
    acc = jnp.dot(x_ref[...], wroot_ref[...],
                  preferred_element_type=jnp.float32) + b_ref[...]
    cnt = jnp.maximum(jnp.sum(c_ref[...], axis=1), 1.0)
    for r in range(N_REL):
        inv_r = 1.0 / cnt[r]
        mean_r = s_ref[r] * lax.broadcast_in_dim(inv_r, s_ref.shape[1:], (0,))
        acc = acc + jnp.dot(mean_r, wrel_ref[r],
                            preferred_element_type=jnp.float32)
    o_ref[...] = jnp.maximum(acc, 0.0) if relu else acc


def _tc_layer(S, C, x, W_rel, W_root, b, relu):
    k = W_rel.shape[2]
    blk = 1280
    grid = (NP // blk,)
    return pl.pallas_call(
        functools.partial(_tc_layer_body, relu=relu),
        grid=grid,
        in_specs=[
            pl.BlockSpec((N_REL, blk, D), lambda i: (0, i, 0)),
            pl.BlockSpec((N_REL, NS, blk), lambda i: (0, 0, i)),
            pl.BlockSpec((blk, D), lambda i: (i, 0)),
            pl.BlockSpec((N_REL, D, k), lambda i: (0, 0, 0)),
            pl.BlockSpec((D, k), lambda i: (0, 0)),
            pl.BlockSpec((1, k), lambda i: (0, 0)),
        ],
        out_specs=pl.BlockSpec((blk, k), lambda i: (i, 0)),
        out_shape=jax.ShapeDtypeStruct((NP, k), jnp.float32),
    )(S, C, x, W_rel, W_root, b)


def kernel(classic_features, edge_index, edge_type, W1_rel, W1_root, b1,
           W2_rel, W2_root, b2):
    src = edge_index[0].astype(jnp.int32)
    dst = edge_index[1].astype(jnp.int32)
    typ = edge_type.astype(jnp.int32)
    edges = src | (dst << 14) | (typ << 28)

    xp = jnp.zeros((NP, D), jnp.float32).at[:N_NODES].set(classic_features)
    S1, C1, LS, LD, NG = _sc_scatter(xp, edges)
    h = _tc_layer(S1, C1, xp, W1_rel, W1_root, b1.reshape(1, -1), relu=True)
    S2 = _sc_scatter2(h, LS, LD, NG)
    out = _tc_layer(S2, C1, h, W2_rel, W2_root, b2.reshape(1, -1),
                    relu=False)
    return out[:N_NODES]


# speedup vs baseline: 17.4950x; 1.0438x over previous
import functools

import jax
import jax.numpy as jnp
from jax import lax
from jax.experimental import pallas as pl
from jax.experimental.pallas import tpu as pltpu
from jax.experimental.pallas import tpu_sc as plsc

N_NODES = 10000
N_EDGES = 320000
D = 128
N_REL = 4
NC = 2
NS = 16
NP = 10240
ROWS_PT = NP // NS
DUMMY = N_NODES + 8
EPT = N_EDGES // NS
CH = 2000
NVR = CH // 16
NCHUNK = EPT // CH
GB = 64
LGB = 6
NR2 = 40
MAXR = 360


def _make_drain(x_hbm, acc_sh, isrc, idst, bufs, gsems, ssems):
    def start_g(base, j, b):
        pltpu.async_copy(x_hbm.at[isrc.at[base + j]], bufs[b], gsems[b])

    def wait_g(b):
        pltpu.make_async_copy(x_hbm.at[isrc.at[0]], bufs[b], gsems[b]).wait()

    def start_s(base, j, b):
        pltpu.async_copy(bufs[b], acc_sh.at[idst.at[base + j]], ssems[b],
                         add=True)

    def wait_s(b):
        pltpu.make_async_copy(bufs[b], acc_sh.at[idst.at[0]], ssems[b]).wait()

    def drain(base, cnt):
        @pl.when(cnt > 0)
        def _():
            start_g(base, 0, 0)

        @pl.when(cnt > 1)
        def _():
            start_g(base, 1, 1)

        def _step(p, _):
            for k in range(3):
                j = 3 * p + k

                @pl.when(j < cnt)
                def _(j=j, k=k):
                    wait_g(k)
                    start_s(base, j, k)

                    @pl.when(j >= 1)
                    def _():
                        wait_s((k + 2) % 3)

                    @pl.when(j + 2 < cnt)
                    def _():
                        start_g(base, j + 2, (k + 2) % 3)

            return 0

        lax.fori_loop(0, (cnt + 2) // 3, _step, 0)
        rem = lax.rem(cnt - 1, 3)
        for k in range(3):
            @pl.when(jnp.logical_and(cnt > 0, rem == k))
            def _(k=k):
                wait_s(k)

    return drain


def _sc_body(x_hbm, edges_hbm, s_hbm, c_hbm,
             lsrc_hbm, ldst_hbm, ngrp_hbm,
             acc_sh, est, csrc, cdst, cnt_v,
             gbuf0, gbuf1, gbuf2, ngbuf,
             gsem0, gsem1, gsem2, ssem0, ssem1, ssem2, psem0, psem1):
    c = lax.axis_index("c")
    s = lax.axis_index("s")
    ebase = s * EPT
    rbase = s * ROWS_PT
    zv = jnp.zeros((16,), jnp.float32)
    ones = jnp.ones((16,), jnp.float32)
    iota16 = lax.iota(jnp.int32, 16)

    _drain = _make_drain(x_hbm, acc_sh, csrc, cdst,
                         (gbuf0, gbuf1, gbuf2),
                         (gsem0, gsem1, gsem2), (ssem0, ssem1, ssem2))

    for rpass in range(N_REL // NC):
        r = c * (N_REL // NC) + rpass

        def _zg(i, _):
            for j in range(D // 16):
                gbuf0[i, pl.ds(j * 16, 16)] = zv
            return 0
        lax.fori_loop(0, GB, _zg, 0)
        for k in range(ROWS_PT // GB):
            pltpu.sync_copy(gbuf0, acc_sh.at[pl.ds(rbase + k * GB, GB)])

        def _zc(i, _):
            cnt_v[pl.ds(i * 16, 16)] = zv
            return 0
        lax.fori_loop(0, NP // 16, _zc, 0)
        plsc.subcore_barrier()

        def _chunk(ci, carry):
            n, pbase = carry
            prow = jnp.right_shift(n, LGB)
            eoff = ebase + ci * CH
            pltpu.sync_copy(edges_hbm.at[pl.ds(eoff, CH)], est)

            def _vec(i, n):
                parts = []
                for q in range(5):
                    w = est[pl.ds(i * 80 + q * 16, 16)]
                    m = jnp.right_shift(w, 28) == r
                    sv = jnp.bitwise_and(w, 16383)
                    dv = jnp.bitwise_and(jnp.right_shift(w, 14), 16383)
                    cum = plsc.cumsum(m.astype(jnp.int32))
                    pc = plsc.all_reduce_population_count(m)[0]
                    parts.append((m, sv, dv, cum, pc))
                for m, sv, dv, cum, pc in parts:
                    pos = n + cum - 1
                    ph = jnp.right_shift(pos, LGB)
                    plo = jnp.bitwise_and(pos, GB - 1)
                    plsc.store_scatter(csrc, [ph, plo], sv, mask=m)
                    plsc.store_scatter(cdst, [ph, plo], dv, mask=m)
                    plsc.addupdate_scatter(cnt_v, [dv], ones, mask=m)
                    n = n + pc
                return n

            n = lax.fori_loop(0, NVR // 5, _vec, n)

            pb8 = pl.multiple_of(pbase, 8)
            pltpu.async_copy(csrc, lsrc_hbm.at[r, s, pl.ds(pb8, NR2)], psem0)
            pltpu.async_copy(cdst, ldst_hbm.at[r, s, pl.ds(pb8, NR2)], psem1)

            nfull = jnp.right_shift(n, LGB)
            _drain(prow, nfull - prow)
            pltpu.make_async_copy(
                csrc, lsrc_hbm.at[r, s, pl.ds(pb8, NR2)], psem0).wait()
            pltpu.make_async_copy(
                cdst, ldst_hbm.at[r, s, pl.ds(pb8, NR2)], psem1).wait()

            pfull8 = jnp.bitwise_and(nfull, ~7)
            for k in range(8):
                for q in range(GB // 16):
                    csrc[k, pl.ds(q * 16, 16)] = (
                        csrc[pfull8 + k, pl.ds(q * 16, 16)])
                    cdst[k, pl.ds(q * 16, 16)] = (
                        cdst[pfull8 + k, pl.ds(q * 16, 16)])
            return n - pfull8 * GB, pbase + pfull8

        n, pbase = lax.fori_loop(0, NCHUNK, _chunk,
                                 (jnp.int32(0), jnp.int32(0)))

        prow = jnp.right_shift(n, LGB)
        for k in range(GB // 16):
            pidx = n + k * 16 + iota16
            ph = jnp.right_shift(pidx, LGB)
            plo = jnp.bitwise_and(pidx, GB - 1)
            plsc.store_scatter(csrc, [ph, plo], jnp.zeros((16,), jnp.int32))
            plsc.store_scatter(cdst, [ph, plo],
                               jnp.full((16,), DUMMY, jnp.int32))
        rows_f = jnp.right_shift(n + GB - 1, LGB)
        pb8 = pl.multiple_of(pbase, 8)
        pltpu.async_copy(csrc.at[pl.ds(0, 16)],
                         lsrc_hbm.at[r, s, pl.ds(pb8, 16)], psem0)
        pltpu.async_copy(cdst.at[pl.ds(0, 16)],
                         ldst_hbm.at[r, s, pl.ds(pb8, 16)], psem1)
        _drain(prow, rows_f - prow)
        pltpu.make_async_copy(
            csrc.at[pl.ds(0, 16)], lsrc_hbm.at[r, s, pl.ds(pb8, 16)],
            psem0).wait()
        pltpu.make_async_copy(
            cdst.at[pl.ds(0, 16)], ldst_hbm.at[r, s, pl.ds(pb8, 16)],
            psem1).wait()

        ngbuf[pl.ds(0, 16)] = jnp.full((16,), 1, jnp.int32) * (pbase + rows_f)
        pltpu.sync_copy(ngbuf, ngrp_hbm.at[r, s])

        plsc.subcore_barrier()
        pltpu.sync_copy(acc_sh.at[pl.ds(rbase, ROWS_PT)],
                        s_hbm.at[r, pl.ds(rbase, ROWS_PT)])
        pltpu.sync_copy(cnt_v, c_hbm.at[r, s])
        plsc.subcore_barrier()


_sc_scatter = pl.kernel(
    _sc_body,
    out_type=(
        jax.ShapeDtypeStruct((N_REL, NP, D), jnp.float32),
        jax.ShapeDtypeStruct((N_REL, NS, NP), jnp.float32),
        jax.ShapeDtypeStruct((N_REL, NS, MAXR, GB), jnp.int32),
        jax.ShapeDtypeStruct((N_REL, NS, MAXR, GB), jnp.int32),
        jax.ShapeDtypeStruct((N_REL, NS, 16), jnp.int32),
    ),
    mesh=plsc.VectorSubcoreMesh(core_axis_name="c", subcore_axis_name="s"),
    compiler_params=pltpu.CompilerParams(needs_layout_passes=False),
    scratch_types=[
        pltpu.VMEM_SHARED((NP, D), jnp.float32),
        pltpu.VMEM((CH,), jnp.int32),
        pltpu.VMEM((NR2, GB), jnp.int32),
        pltpu.VMEM((NR2, GB), jnp.int32),
        pltpu.VMEM((NP,), jnp.float32),
        pltpu.VMEM((GB, D), jnp.float32),
        pltpu.VMEM((GB, D), jnp.float32),
        pltpu.VMEM((GB, D), jnp.float32),
        pltpu.VMEM((16,), jnp.int32),
        pltpu.SemaphoreType.DMA,
        pltpu.SemaphoreType.DMA,
        pltpu.SemaphoreType.DMA,
        pltpu.SemaphoreType.DMA,
        pltpu.SemaphoreType.DMA,
        pltpu.SemaphoreType.DMA,
        pltpu.SemaphoreType.DMA,
        pltpu.SemaphoreType.DMA,
    ],
)


def _sc_body2(x_hbm, lsrc_hbm, ldst_hbm, ngrp_hbm, s_hbm,
              acc_sh, isrc_a, idst_a, isrc_b, idst_b,
              gbuf0, gbuf1, gbuf2, ngbuf,
              gsem0, gsem1, gsem2, ssem0, ssem1, ssem2,
              psa0, psa1, psb0, psb1):
    c = lax.axis_index("c")
    s = lax.axis_index("s")
    rbase = s * ROWS_PT
    zv = jnp.zeros((16,), jnp.float32)

    bufs = (gbuf0, gbuf1, gbuf2)
    gsems = (gsem0, gsem1, gsem2)
    ssems = (ssem0, ssem1, ssem2)
    _drain_a = _make_drain(x_hbm, acc_sh, isrc_a, idst_a, bufs, gsems, ssems)
    _drain_b = _make_drain(x_hbm, acc_sh, isrc_b, idst_b, bufs, gsems, ssems)

    def _stage(blk, isrc, idst, r, sem0, sem1):
        pltpu.async_copy(lsrc_hbm.at[r, s, pl.ds(blk * NR2, NR2)], isrc,
                         sem0)
        pltpu.async_copy(ldst_hbm.at[r, s, pl.ds(blk * NR2, NR2)], idst,
                         sem1)

    def _stage_wait(blk, isrc, idst, r, sem0, sem1):
        pltpu.make_async_copy(lsrc_hbm.at[r, s, pl.ds(blk * NR2, NR2)],
                              isrc, sem0).wait()
        pltpu.make_async_copy(ldst_hbm.at[r, s, pl.ds(blk * NR2, NR2)],
                              idst, sem1).wait()

    for rpass in range(N_REL // NC):
        r = c * (N_REL // NC) + rpass

        def _zg(i, _):
            for j in range(D // 16):
                gbuf0[i, pl.ds(j * 16, 16)] = zv
            return 0
        lax.fori_loop(0, GB, _zg, 0)
        for k in range(ROWS_PT // GB):
            pltpu.sync_copy(gbuf0, acc_sh.at[pl.ds(rbase + k * GB, GB)])
        plsc.subcore_barrier()

        pltpu.sync_copy(ngrp_hbm.at[r, s], ngbuf)
        ng = ngbuf[pl.ds(0, 16)][0]
        nblk = (ng + NR2 - 1) // NR2

        @pl.when(nblk > 0)
        def _():
            _stage(0, isrc_a, idst_a, r, psa0, psa1)
            _stage_wait(0, isrc_a, idst_a, r, psa0, psa1)

        def _pstep(p, _):
            k0 = 2 * p
            k1 = k0 + 1

            @pl.when(k1 < nblk)
            def _():
                _stage(k1, isrc_b, idst_b, r, psb0, psb1)

            @pl.when(k0 >= 1)
            def _():
                _stage_wait(k0, isrc_a, idst_a, r, psa0, psa1)

            _drain_a(0, jnp.minimum(ng - k0 * NR2, NR2))

            @pl.when(k1 < nblk)
            def _():
                @pl.when(k1 + 1 < nblk)
                def _():
                    _stage(k1 + 1, isrc_a, idst_a, r, psa0, psa1)

                _stage_wait(k1, isrc_b, idst_b, r, psb0, psb1)
                _drain_b(0, jnp.minimum(ng - k1 * NR2, NR2))

            return 0

        lax.fori_loop(0, (nblk + 1) >> 1, _pstep, 0)

        plsc.subcore_barrier()
        pltpu.sync_copy(acc_sh.at[pl.ds(rbase, ROWS_PT)],
                        s_hbm.at[r, pl.ds(rbase, ROWS_PT)])
        plsc.subcore_barrier()


_sc_scatter2 = pl.kernel(
    _sc_body2,
    out_type=jax.ShapeDtypeStruct((N_REL, NP, D), jnp.float32),
    mesh=plsc.VectorSubcoreMesh(core_axis_name="c", subcore_axis_name="s"),
    compiler_params=pltpu.CompilerParams(needs_layout_passes=False),
    scratch_types=[
        pltpu.VMEM_SHARED((NP, D), jnp.float32),
        pltpu.VMEM((NR2, GB), jnp.int32),
        pltpu.VMEM((NR2, GB), jnp.int32),
        pltpu.VMEM((NR2, GB), jnp.int32),
        pltpu.VMEM((NR2, GB), jnp.int32),
        pltpu.VMEM((GB, D), jnp.float32),
        pltpu.VMEM((GB, D), jnp.float32),
        pltpu.VMEM((GB, D), jnp.float32),
        pltpu.VMEM((16,), jnp.int32),
        pltpu.SemaphoreType.DMA,
        pltpu.SemaphoreType.DMA,
        pltpu.SemaphoreType.DMA,
        pltpu.SemaphoreType.DMA,
        pltpu.SemaphoreType.DMA,
        pltpu.SemaphoreType.DMA,
        pltpu.SemaphoreType.DMA,
        pltpu.SemaphoreType.DMA,
        pltpu.SemaphoreType.DMA,
        pltpu.SemaphoreType.DMA,
    ],
)


def _tc_layer_body(s_ref, c_ref, x_ref, wrel_ref, wroot_ref, b_ref, o_ref,
                   *, relu):
    acc = jnp.dot(x_ref[...], wroot_ref[...],
                  preferred_element_type=jnp.float32) + b_ref[...]
    cnt = jnp.maximum(jnp.sum(c_ref[...], axis=1), 1.0)
    for r in range(N_REL):
        inv_r = 1.0 / cnt[r]
        mean_r = s_ref[r] * lax.broadcast_in_dim(inv_r, s_ref.shape[1:], (0,))
        acc = acc + jnp.dot(mean_r, wrel_ref[r],
                            preferred_element_type=jnp.float32)
    o_ref[...] = jnp.maximum(acc, 0.0) if relu else acc


def _tc_layer(S, C, x, W_rel, W_root, b, relu):
    k = W_rel.shape[2]
    blk = 1280
    grid = (NP // blk,)
    return pl.pallas_call(
        functools.partial(_tc_layer_body, relu=relu),
        grid=grid,
        in_specs=[
            pl.BlockSpec((N_REL, blk, D), lambda i: (0, i, 0)),
            pl.BlockSpec((N_REL, NS, blk), lambda i: (0, 0, i)),
            pl.BlockSpec((blk, D), lambda i: (i, 0)),
            pl.BlockSpec((N_REL, D, k), lambda i: (0, 0, 0)),
            pl.BlockSpec((D, k), lambda i: (0, 0)),
            pl.BlockSpec((1, k), lambda i: (0, 0)),
        ],
        out_specs=pl.BlockSpec((blk, k), lambda i: (i, 0)),
        out_shape=jax.ShapeDtypeStruct((NP, k), jnp.float32),
    )(S, C, x, W_rel, W_root, b)


def kernel(classic_features, edge_index, edge_type, W1_rel, W1_root, b1,
           W2_rel, W2_root, b2):
    src = edge_index[0].astype(jnp.int32)
    dst = edge_index[1].astype(jnp.int32)
    typ = edge_type.astype(jnp.int32)
    edges = src | (dst << 14) | (typ << 28)

    xp = jnp.zeros((NP, D), jnp.float32).at[:N_NODES].set(classic_features)
    S1, C1, LS, LD, NG = _sc_scatter(xp, edges)
    h = _tc_layer(S1, C1, xp, W1_rel, W1_root, b1.reshape(1, -1), relu=True)
    S2 = _sc_scatter2(h, LS, LD, NG)
    out = _tc_layer(S2, C1, h, W2_rel, W2_root, b2.reshape(1, -1),
                    relu=False)
    return out[:N_NODES]


# speedup vs baseline: 17.7669x; 1.0155x over previous
import functools

import jax
import jax.numpy as jnp
from jax import lax
from jax.experimental import pallas as pl
from jax.experimental.pallas import tpu as pltpu
from jax.experimental.pallas import tpu_sc as plsc

N_NODES = 10000
N_EDGES = 320000
D = 128
N_REL = 4
NC = 2
NS = 16
NP = 10240
ROWS_PT = NP // NS
DUMMY = N_NODES + 8
EPT = N_EDGES // NS
CH = 2000
NVR = CH // 16
NCHUNK = EPT // CH
GB = 64
LGB = 6
NR2 = 40
MAXR = 360


def _make_drain(x_hbm, acc_sh, isrc, idst, bufs, gsems, ssems):
    def start_g(base, j, b):
        pltpu.async_copy(x_hbm.at[isrc.at[base + j]], bufs[b], gsems[b])

    def wait_g(b):
        pltpu.make_async_copy(x_hbm.at[isrc.at[0]], bufs[b], gsems[b]).wait()

    def start_s(base, j, b):
        pltpu.async_copy(bufs[b], acc_sh.at[idst.at[base + j]], ssems[b],
                         add=True)

    def wait_s(b):
        pltpu.make_async_copy(bufs[b], acc_sh.at[idst.at[0]], ssems[b]).wait()

    def drain(base, cnt):
        @pl.when(cnt > 0)
        def _():
            start_g(base, 0, 0)

        @pl.when(cnt > 1)
        def _():
            start_g(base, 1, 1)

        def _step(p, _):
            for k in range(3):
                j = 3 * p + k

                @pl.when(j < cnt)
                def _(j=j, k=k):
                    wait_g(k)
                    start_s(base, j, k)

                    @pl.when(j >= 1)
                    def _():
                        wait_s((k + 2) % 3)

                    @pl.when(j + 2 < cnt)
                    def _():
                        start_g(base, j + 2, (k + 2) % 3)

            return 0

        lax.fori_loop(0, (cnt + 2) // 3, _step, 0)
        rem = lax.rem(cnt - 1, 3)
        for k in range(3):
            @pl.when(jnp.logical_and(cnt > 0, rem == k))
            def _(k=k):
                wait_s(k)

    return drain


def _sc_body(x_hbm, edges_hbm, s_hbm, c_hbm,
             lsrc_hbm, ldst_hbm, ngrp_hbm,
             acc_sh, est, csrc, cdst, cnt_v,
             gbuf0, gbuf1, gbuf2, ngbuf,
             gsem0, gsem1, gsem2, ssem0, ssem1, ssem2, psem0, psem1):
    c = lax.axis_index("c")
    s = lax.axis_index("s")
    ebase = s * EPT
    rbase = s * ROWS_PT
    zv = jnp.zeros((16,), jnp.float32)
    ones = jnp.ones((16,), jnp.float32)
    iota16 = lax.iota(jnp.int32, 16)

    _drain = _make_drain(x_hbm, acc_sh, csrc, cdst,
                         (gbuf0, gbuf1, gbuf2),
                         (gsem0, gsem1, gsem2), (ssem0, ssem1, ssem2))

    for rpass in range(N_REL // NC):
        r = c * (N_REL // NC) + rpass

        def _zg(i, _):
            for j in range(D // 16):
                gbuf0[i, pl.ds(j * 16, 16)] = zv
            return 0
        lax.fori_loop(0, GB, _zg, 0)
        for k in range(ROWS_PT // GB):
            pltpu.async_copy(gbuf0, acc_sh.at[pl.ds(rbase + k * GB, GB)],
                             psem0)

        def _zc(i, _):
            cnt_v[pl.ds(i * 16, 16)] = zv
            return 0
        lax.fori_loop(0, NP // 16, _zc, 0)
        for k in range(ROWS_PT // GB):
            pltpu.make_async_copy(
                gbuf0, acc_sh.at[pl.ds(rbase + k * GB, GB)], psem0).wait()
        plsc.subcore_barrier()

        def _chunk(ci, carry):
            n, pbase = carry
            prow = jnp.right_shift(n, LGB)
            eoff = ebase + ci * CH
            pltpu.sync_copy(edges_hbm.at[pl.ds(eoff, CH)], est)

            def _vec(i, n):
                parts = []
                for q in range(5):
                    w = est[pl.ds(i * 80 + q * 16, 16)]
                    m = jnp.right_shift(w, 28) == r
                    sv = jnp.bitwise_and(w, 16383)
                    dv = jnp.bitwise_and(jnp.right_shift(w, 14), 16383)
                    cum = plsc.cumsum(m.astype(jnp.int32))
                    pc = plsc.all_reduce_population_count(m)[0]
                    parts.append((m, sv, dv, cum, pc))
                for m, sv, dv, cum, pc in parts:
                    pos = n + cum - 1
                    ph = jnp.right_shift(pos, LGB)
                    plo = jnp.bitwise_and(pos, GB - 1)
                    plsc.store_scatter(csrc, [ph, plo], sv, mask=m)
                    plsc.store_scatter(cdst, [ph, plo], dv, mask=m)
                    plsc.addupdate_scatter(cnt_v, [dv], ones, mask=m)
                    n = n + pc
                return n

            n = lax.fori_loop(0, NVR // 5, _vec, n)

            pb8 = pl.multiple_of(pbase, 8)
            pltpu.async_copy(csrc, lsrc_hbm.at[r, s, pl.ds(pb8, NR2)], psem0)
            pltpu.async_copy(cdst, ldst_hbm.at[r, s, pl.ds(pb8, NR2)], psem1)

            nfull = jnp.right_shift(n, LGB)
            _drain(prow, nfull - prow)
            pltpu.make_async_copy(
                csrc, lsrc_hbm.at[r, s, pl.ds(pb8, NR2)], psem0).wait()
            pltpu.make_async_copy(
                cdst, ldst_hbm.at[r, s, pl.ds(pb8, NR2)], psem1).wait()

            pfull8 = jnp.bitwise_and(nfull, ~7)
            for k in range(8):
                for q in range(GB // 16):
                    csrc[k, pl.ds(q * 16, 16)] = (
                        csrc[pfull8 + k, pl.ds(q * 16, 16)])
                    cdst[k, pl.ds(q * 16, 16)] = (
                        cdst[pfull8 + k, pl.ds(q * 16, 16)])
            return n - pfull8 * GB, pbase + pfull8

        n, pbase = lax.fori_loop(0, NCHUNK, _chunk,
                                 (jnp.int32(0), jnp.int32(0)))

        prow = jnp.right_shift(n, LGB)
        for k in range(GB // 16):
            pidx = n + k * 16 + iota16
            ph = jnp.right_shift(pidx, LGB)
            plo = jnp.bitwise_and(pidx, GB - 1)
            plsc.store_scatter(csrc, [ph, plo], jnp.zeros((16,), jnp.int32))
            plsc.store_scatter(cdst, [ph, plo],
                               jnp.full((16,), DUMMY, jnp.int32))
        rows_f = jnp.right_shift(n + GB - 1, LGB)
        pb8 = pl.multiple_of(pbase, 8)
        pltpu.async_copy(csrc.at[pl.ds(0, 16)],
                         lsrc_hbm.at[r, s, pl.ds(pb8, 16)], psem0)
        pltpu.async_copy(cdst.at[pl.ds(0, 16)],
                         ldst_hbm.at[r, s, pl.ds(pb8, 16)], psem1)
        _drain(prow, rows_f - prow)
        pltpu.make_async_copy(
            csrc.at[pl.ds(0, 16)], lsrc_hbm.at[r, s, pl.ds(pb8, 16)],
            psem0).wait()
        pltpu.make_async_copy(
            cdst.at[pl.ds(0, 16)], ldst_hbm.at[r, s, pl.ds(pb8, 16)],
            psem1).wait()

        ngbuf[pl.ds(0, 16)] = jnp.full((16,), 1, jnp.int32) * (pbase + rows_f)
        pltpu.sync_copy(ngbuf, ngrp_hbm.at[r, s])

        plsc.subcore_barrier()
        pltpu.sync_copy(acc_sh.at[pl.ds(rbase, ROWS_PT)],
                        s_hbm.at[r, pl.ds(rbase, ROWS_PT)])
        pltpu.sync_copy(cnt_v, c_hbm.at[r, s])
        plsc.subcore_barrier()


_sc_scatter = pl.kernel(
    _sc_body,
    out_type=(
        jax.ShapeDtypeStruct((N_REL, NP, D), jnp.float32),
        jax.ShapeDtypeStruct((N_REL, NS, NP), jnp.float32),
        jax.ShapeDtypeStruct((N_REL, NS, MAXR, GB), jnp.int32),
        jax.ShapeDtypeStruct((N_REL, NS, MAXR, GB), jnp.int32),
        jax.ShapeDtypeStruct((N_REL, NS, 16), jnp.int32),
    ),
    mesh=plsc.VectorSubcoreMesh(core_axis_name="c", subcore_axis_name="s"),
    compiler_params=pltpu.CompilerParams(needs_layout_passes=False),
    scratch_types=[
        pltpu.VMEM_SHARED((NP, D), jnp.float32),
        pltpu.VMEM((CH,), jnp.int32),
        pltpu.VMEM((NR2, GB), jnp.int32),
        pltpu.VMEM((NR2, GB), jnp.int32),
        pltpu.VMEM((NP,), jnp.float32),
        pltpu.VMEM((GB, D), jnp.float32),
        pltpu.VMEM((GB, D), jnp.float32),
        pltpu.VMEM((GB, D), jnp.float32),
        pltpu.VMEM((16,), jnp.int32),
        pltpu.SemaphoreType.DMA,
        pltpu.SemaphoreType.DMA,
        pltpu.SemaphoreType.DMA,
        pltpu.SemaphoreType.DMA,
        pltpu.SemaphoreType.DMA,
        pltpu.SemaphoreType.DMA,
        pltpu.SemaphoreType.DMA,
        pltpu.SemaphoreType.DMA,
    ],
)


def _sc_body2(x_hbm, lsrc_hbm, ldst_hbm, ngrp_hbm, s_hbm,
              acc_sh, isrc_a, idst_a, isrc_b, idst_b,
              gbuf0, gbuf1, gbuf2, ngbuf,
              gsem0, gsem1, gsem2, ssem0, ssem1, ssem2,
              psa0, psa1, psb0, psb1):
    c = lax.axis_index("c")
    s = lax.axis_index("s")
    rbase = s * ROWS_PT
    zv = jnp.zeros((16,), jnp.float32)

    bufs = (gbuf0, gbuf1, gbuf2)
    gsems = (gsem0, gsem1, gsem2)
    ssems = (ssem0, ssem1, ssem2)
    _drain_a = _make_drain(x_hbm, acc_sh, isrc_a, idst_a, bufs, gsems, ssems)
    _drain_b = _make_drain(x_hbm, acc_sh, isrc_b, idst_b, bufs, gsems, ssems)

    def _stage(blk, isrc, idst, r, sem0, sem1):
        pltpu.async_copy(lsrc_hbm.at[r, s, pl.ds(blk * NR2, NR2)], isrc,
                         sem0)
        pltpu.async_copy(ldst_hbm.at[r, s, pl.ds(blk * NR2, NR2)], idst,
                         sem1)

    def _stage_wait(blk, isrc, idst, r, sem0, sem1):
        pltpu.make_async_copy(lsrc_hbm.at[r, s, pl.ds(blk * NR2, NR2)],
                              isrc, sem0).wait()
        pltpu.make_async_copy(ldst_hbm.at[r, s, pl.ds(blk * NR2, NR2)],
                              idst, sem1).wait()

    for rpass in range(N_REL // NC):
        r = c * (N_REL // NC) + rpass

        def _zg(i, _):
            for j in range(D // 16):
                gbuf0[i, pl.ds(j * 16, 16)] = zv
            return 0
        lax.fori_loop(0, GB, _zg, 0)
        for k in range(ROWS_PT // GB):
            pltpu.async_copy(gbuf0, acc_sh.at[pl.ds(rbase + k * GB, GB)],
                             psa0)
        for k in range(ROWS_PT // GB):
            pltpu.make_async_copy(
                gbuf0, acc_sh.at[pl.ds(rbase + k * GB, GB)], psa0).wait()
        plsc.subcore_barrier()

        pltpu.sync_copy(ngrp_hbm.at[r, s], ngbuf)
        ng = ngbuf[pl.ds(0, 16)][0]
        nblk = (ng + NR2 - 1) // NR2

        @pl.when(nblk > 0)
        def _():
            _stage(0, isrc_a, idst_a, r, psa0, psa1)
            _stage_wait(0, isrc_a, idst_a, r, psa0, psa1)

        def _pstep(p, _):
            k0 = 2 * p
            k1 = k0 + 1

            @pl.when(k1 < nblk)
            def _():
                _stage(k1, isrc_b, idst_b, r, psb0, psb1)

            @pl.when(k0 >= 1)
            def _():
                _stage_wait(k0, isrc_a, idst_a, r, psa0, psa1)

            _drain_a(0, jnp.minimum(ng - k0 * NR2, NR2))

            @pl.when(k1 < nblk)
            def _():
                @pl.when(k1 + 1 < nblk)
                def _():
                    _stage(k1 + 1, isrc_a, idst_a, r, psa0, psa1)

                _stage_wait(k1, isrc_b, idst_b, r, psb0, psb1)
                _drain_b(0, jnp.minimum(ng - k1 * NR2, NR2))

            return 0

        lax.fori_loop(0, (nblk + 1) >> 1, _pstep, 0)

        plsc.subcore_barrier()
        pltpu.sync_copy(acc_sh.at[pl.ds(rbase, ROWS_PT)],
                        s_hbm.at[r, pl.ds(rbase, ROWS_PT)])
        plsc.subcore_barrier()


_sc_scatter2 = pl.kernel(
    _sc_body2,
    out_type=jax.ShapeDtypeStruct((N_REL, NP, D), jnp.float32),
    mesh=plsc.VectorSubcoreMesh(core_axis_name="c", subcore_axis_name="s"),
    compiler_params=pltpu.CompilerParams(needs_layout_passes=False),
    scratch_types=[
        pltpu.VMEM_SHARED((NP, D), jnp.float32),
        pltpu.VMEM((NR2, GB), jnp.int32),
        pltpu.VMEM((NR2, GB), jnp.int32),
        pltpu.VMEM((NR2, GB), jnp.int32),
        pltpu.VMEM((NR2, GB), jnp.int32),
        pltpu.VMEM((GB, D), jnp.float32),
        pltpu.VMEM((GB, D), jnp.float32),
        pltpu.VMEM((GB, D), jnp.float32),
        pltpu.VMEM((16,), jnp.int32),
        pltpu.SemaphoreType.DMA,
        pltpu.SemaphoreType.DMA,
        pltpu.SemaphoreType.DMA,
        pltpu.SemaphoreType.DMA,
        pltpu.SemaphoreType.DMA,
        pltpu.SemaphoreType.DMA,
        pltpu.SemaphoreType.DMA,
        pltpu.SemaphoreType.DMA,
        pltpu.SemaphoreType.DMA,
        pltpu.SemaphoreType.DMA,
    ],
)


def _tc_layer_body(s_ref, c_ref, x_ref, wrel_ref, wroot_ref, b_ref, o_ref,
                   *, relu):
    acc = jnp.dot(x_ref[...], wroot_ref[...],
                  preferred_element_type=jnp.float32) + b_ref[...]
    cnt = jnp.maximum(jnp.sum(c_ref[...], axis=1), 1.0)
    for r in range(N_REL):
        inv_r = 1.0 / cnt[r]
        mean_r = s_ref[r] * lax.broadcast_in_dim(inv_r, s_ref.shape[1:], (0,))
        acc = acc + jnp.dot(mean_r, wrel_ref[r],
                            preferred_element_type=jnp.float32)
    o_ref[...] = jnp.maximum(acc, 0.0) if relu else acc


def _tc_layer(S, C, x, W_rel, W_root, b, relu):
    k = W_rel.shape[2]
    blk = 1280
    grid = (NP // blk,)
    return pl.pallas_call(
        functools.partial(_tc_layer_body, relu=relu),
        grid=grid,
        in_specs=[
            pl.BlockSpec((N_REL, blk, D), lambda i: (0, i, 0)),
            pl.BlockSpec((N_REL, NS, blk), lambda i: (0, 0, i)),
            pl.BlockSpec((blk, D), lambda i: (i, 0)),
            pl.BlockSpec((N_REL, D, k), lambda i: (0, 0, 0)),
            pl.BlockSpec((D, k), lambda i: (0, 0)),
            pl.BlockSpec((1, k), lambda i: (0, 0)),
        ],
        out_specs=pl.BlockSpec((blk, k), lambda i: (i, 0)),
        out_shape=jax.ShapeDtypeStruct((NP, k), jnp.float32),
    )(S, C, x, W_rel, W_root, b)


def kernel(classic_features, edge_index, edge_type, W1_rel, W1_root, b1,
           W2_rel, W2_root, b2):
    src = edge_index[0].astype(jnp.int32)
    dst = edge_index[1].astype(jnp.int32)
    typ = edge_type.astype(jnp.int32)
    edges = src | (dst << 14) | (typ << 28)

    xp = jnp.zeros((NP, D), jnp.float32).at[:N_NODES].set(classic_features)
    S1, C1, LS, LD, NG = _sc_scatter(xp, edges)
    h = _tc_layer(S1, C1, xp, W1_rel, W1_root, b1.reshape(1, -1), relu=True)
    S2 = _sc_scatter2(h, LS, LD, NG)
    out = _tc_layer(S2, C1, h, W2_rel, W2_root, b2.reshape(1, -1),
                    relu=False)
    return out[:N_NODES]


# speedup vs baseline: 17.9834x; 1.0122x over previous
import functools

import jax
import jax.numpy as jnp
from jax import lax
from jax.experimental import pallas as pl
from jax.experimental.pallas import tpu as pltpu
from jax.experimental.pallas import tpu_sc as plsc

N_NODES = 10000
N_EDGES = 320000
D = 128
N_REL = 4
NC = 2
NS = 16
NP = 10240
ROWS_PT = NP // NS
DUMMY = N_NODES + 8
EPT = N_EDGES // NS
CH = 2000
NVR = CH // 16
NCHUNK = EPT // CH
GB = 64
LGB = 6
NR2 = 40
MAXR = 360


def _make_drain(x_hbm, acc_sh, isrc, idst, bufs, gsems, ssems):
    def start_g(base, j, b):
        pltpu.async_copy(x_hbm.at[isrc.at[base + j]], bufs[b], gsems[b])

    def wait_g(b):
        pltpu.make_async_copy(x_hbm.at[isrc.at[0]], bufs[b], gsems[b]).wait()

    def start_s(base, j, b):
        pltpu.async_copy(bufs[b], acc_sh.at[idst.at[base + j]], ssems[b],
                         add=True)

    def wait_s(b):
        pltpu.make_async_copy(bufs[b], acc_sh.at[idst.at[0]], ssems[b]).wait()

    def drain(base, cnt):
        @pl.when(cnt > 0)
        def _():
            start_g(base, 0, 0)

        @pl.when(cnt > 1)
        def _():
            start_g(base, 1, 1)

        def _step(p, _):
            for k in range(3):
                j = 3 * p + k

                @pl.when(j < cnt)
                def _(j=j, k=k):
                    wait_g(k)
                    start_s(base, j, k)

                    @pl.when(j >= 1)
                    def _():
                        wait_s((k + 2) % 3)

                    @pl.when(j + 2 < cnt)
                    def _():
                        start_g(base, j + 2, (k + 2) % 3)

            return 0

        lax.fori_loop(0, (cnt + 2) // 3, _step, 0)
        rem = lax.rem(cnt - 1, 3)
        for k in range(3):
            @pl.when(jnp.logical_and(cnt > 0, rem == k))
            def _(k=k):
                wait_s(k)

    return drain


def _sc_body(x_hbm, edges_hbm, s_hbm, c_hbm,
             lsrc_hbm, ldst_hbm, ngrp_hbm,
             acc_sh, est_a, est_b, csrc, cdst, cnt_v,
             gbuf0, gbuf1, gbuf2, ngbuf,
             gsem0, gsem1, gsem2, ssem0, ssem1, ssem2, psem0, psem1,
             esema, esemb):
    c = lax.axis_index("c")
    s = lax.axis_index("s")
    ebase = s * EPT
    rbase = s * ROWS_PT
    zv = jnp.zeros((16,), jnp.float32)
    ones = jnp.ones((16,), jnp.float32)
    iota16 = lax.iota(jnp.int32, 16)

    _drain = _make_drain(x_hbm, acc_sh, csrc, cdst,
                         (gbuf0, gbuf1, gbuf2),
                         (gsem0, gsem1, gsem2), (ssem0, ssem1, ssem2))

    for rpass in range(N_REL // NC):
        r = c * (N_REL // NC) + rpass

        def _zg(i, _):
            for j in range(D // 16):
                gbuf0[i, pl.ds(j * 16, 16)] = zv
            return 0
        lax.fori_loop(0, GB, _zg, 0)
        for k in range(ROWS_PT // GB):
            pltpu.async_copy(gbuf0, acc_sh.at[pl.ds(rbase + k * GB, GB)],
                             psem0)

        def _zc(i, _):
            cnt_v[pl.ds(i * 16, 16)] = zv
            return 0
        lax.fori_loop(0, N_NODES // 16, _zc, 0)
        for k in range(ROWS_PT // GB):
            pltpu.make_async_copy(
                gbuf0, acc_sh.at[pl.ds(rbase + k * GB, GB)], psem0).wait()
        plsc.subcore_barrier()

        def _chunk(est, ci, carry):
            n, pbase = carry
            prow = jnp.right_shift(n, LGB)

            def _vec(i, n):
                parts = []
                for q in range(5):
                    w = est[pl.ds(i * 80 + q * 16, 16)]
                    m = jnp.right_shift(w, 28) == r
                    sv = jnp.bitwise_and(w, 16383)
                    dv = jnp.bitwise_and(jnp.right_shift(w, 14), 16383)
                    cum = plsc.cumsum(m.astype(jnp.int32))
                    pc = plsc.all_reduce_population_count(m)[0]
                    parts.append((m, sv, dv, cum, pc))
                for m, sv, dv, cum, pc in parts:
                    pos = n + cum - 1
                    ph = jnp.right_shift(pos, LGB)
                    plo = jnp.bitwise_and(pos, GB - 1)
                    plsc.store_scatter(csrc, [ph, plo], sv, mask=m)
                    plsc.store_scatter(cdst, [ph, plo], dv, mask=m)
                    plsc.addupdate_scatter(cnt_v, [dv], ones, mask=m)
                    n = n + pc
                return n

            n = lax.fori_loop(0, NVR // 5, _vec, n)

            pb8 = pl.multiple_of(pbase, 8)
            pltpu.async_copy(csrc, lsrc_hbm.at[r, s, pl.ds(pb8, NR2)], psem0)
            pltpu.async_copy(cdst, ldst_hbm.at[r, s, pl.ds(pb8, NR2)], psem1)

            nfull = jnp.right_shift(n, LGB)
            _drain(prow, nfull - prow)
            pltpu.make_async_copy(
                csrc, lsrc_hbm.at[r, s, pl.ds(pb8, NR2)], psem0).wait()
            pltpu.make_async_copy(
                cdst, ldst_hbm.at[r, s, pl.ds(pb8, NR2)], psem1).wait()

            pfull8 = jnp.bitwise_and(nfull, ~7)
            for k in range(8):
                for q in range(GB // 16):
                    csrc[k, pl.ds(q * 16, 16)] = (
                        csrc[pfull8 + k, pl.ds(q * 16, 16)])
                    cdst[k, pl.ds(q * 16, 16)] = (
                        cdst[pfull8 + k, pl.ds(q * 16, 16)])
            return n - pfull8 * GB, pbase + pfull8

        def _estage(ci, est, sem):
            pltpu.async_copy(edges_hbm.at[pl.ds(ebase + ci * CH, CH)], est,
                             sem)

        def _estage_wait(est, sem):
            pltpu.make_async_copy(edges_hbm.at[pl.ds(ebase, CH)], est,
                                  sem).wait()

        pltpu.sync_copy(edges_hbm.at[pl.ds(ebase, CH)], est_a)

        def _cpair(q, carry):
            c0 = 2 * q
            c1 = c0 + 1
            _estage(c1, est_b, esemb)

            @pl.when(q >= 1)
            def _():
                _estage_wait(est_a, esema)

            carry = _chunk(est_a, c0, carry)
            _estage_wait(est_b, esemb)

            @pl.when(c1 + 1 < NCHUNK)
            def _():
                _estage(c1 + 1, est_a, esema)

            carry = _chunk(est_b, c1, carry)
            return carry

        n, pbase = lax.fori_loop(0, NCHUNK // 2, _cpair,
                                 (jnp.int32(0), jnp.int32(0)))

        prow = jnp.right_shift(n, LGB)
        for k in range(GB // 16):
            pidx = n + k * 16 + iota16
            ph = jnp.right_shift(pidx, LGB)
            plo = jnp.bitwise_and(pidx, GB - 1)
            plsc.store_scatter(csrc, [ph, plo], jnp.zeros((16,), jnp.int32))
            plsc.store_scatter(cdst, [ph, plo],
                               jnp.full((16,), DUMMY, jnp.int32))
        rows_f = jnp.right_shift(n + GB - 1, LGB)
        pb8 = pl.multiple_of(pbase, 8)
        pltpu.async_copy(csrc.at[pl.ds(0, 16)],
                         lsrc_hbm.at[r, s, pl.ds(pb8, 16)], psem0)
        pltpu.async_copy(cdst.at[pl.ds(0, 16)],
                         ldst_hbm.at[r, s, pl.ds(pb8, 16)], psem1)
        _drain(prow, rows_f - prow)
        pltpu.make_async_copy(
            csrc.at[pl.ds(0, 16)], lsrc_hbm.at[r, s, pl.ds(pb8, 16)],
            psem0).wait()
        pltpu.make_async_copy(
            cdst.at[pl.ds(0, 16)], ldst_hbm.at[r, s, pl.ds(pb8, 16)],
            psem1).wait()

        ngbuf[pl.ds(0, 16)] = jnp.full((16,), 1, jnp.int32) * (pbase + rows_f)
        pltpu.sync_copy(ngbuf, ngrp_hbm.at[r, s])

        plsc.subcore_barrier()
        pltpu.sync_copy(acc_sh.at[pl.ds(rbase, ROWS_PT)],
                        s_hbm.at[r, pl.ds(rbase, ROWS_PT)])
        pltpu.sync_copy(cnt_v, c_hbm.at[r, s])
        plsc.subcore_barrier()


_sc_scatter = pl.kernel(
    _sc_body,
    out_type=(
        jax.ShapeDtypeStruct((N_REL, NP, D), jnp.float32),
        jax.ShapeDtypeStruct((N_REL, NS, N_NODES), jnp.float32),
        jax.ShapeDtypeStruct((N_REL, NS, MAXR, GB), jnp.int32),
        jax.ShapeDtypeStruct((N_REL, NS, MAXR, GB), jnp.int32),
        jax.ShapeDtypeStruct((N_REL, NS, 16), jnp.int32),
    ),
    mesh=plsc.VectorSubcoreMesh(core_axis_name="c", subcore_axis_name="s"),
    compiler_params=pltpu.CompilerParams(needs_layout_passes=False),
    scratch_types=[
        pltpu.VMEM_SHARED((NP, D), jnp.float32),
        pltpu.VMEM((CH,), jnp.int32),
        pltpu.VMEM((CH,), jnp.int32),
        pltpu.VMEM((NR2, GB), jnp.int32),
        pltpu.VMEM((NR2, GB), jnp.int32),
        pltpu.VMEM((N_NODES,), jnp.float32),
        pltpu.VMEM((GB, D), jnp.float32),
        pltpu.VMEM((GB, D), jnp.float32),
        pltpu.VMEM((GB, D), jnp.float32),
        pltpu.VMEM((16,), jnp.int32),
        pltpu.SemaphoreType.DMA,
        pltpu.SemaphoreType.DMA,
        pltpu.SemaphoreType.DMA,
        pltpu.SemaphoreType.DMA,
        pltpu.SemaphoreType.DMA,
        pltpu.SemaphoreType.DMA,
        pltpu.SemaphoreType.DMA,
        pltpu.SemaphoreType.DMA,
        pltpu.SemaphoreType.DMA,
        pltpu.SemaphoreType.DMA,
    ],
)


def _sc_body2(x_hbm, lsrc_hbm, ldst_hbm, ngrp_hbm, s_hbm,
              acc_sh, isrc_a, idst_a, isrc_b, idst_b,
              gbuf0, gbuf1, gbuf2, ngbuf,
              gsem0, gsem1, gsem2, ssem0, ssem1, ssem2,
              psa0, psa1, psb0, psb1):
    c = lax.axis_index("c")
    s = lax.axis_index("s")
    rbase = s * ROWS_PT
    zv = jnp.zeros((16,), jnp.float32)

    bufs = (gbuf0, gbuf1, gbuf2)
    gsems = (gsem0, gsem1, gsem2)
    ssems = (ssem0, ssem1, ssem2)
    _drain_a = _make_drain(x_hbm, acc_sh, isrc_a, idst_a, bufs, gsems, ssems)
    _drain_b = _make_drain(x_hbm, acc_sh, isrc_b, idst_b, bufs, gsems, ssems)

    def _stage(blk, isrc, idst, r, sem0, sem1):
        pltpu.async_copy(lsrc_hbm.at[r, s, pl.ds(blk * NR2, NR2)], isrc,
                         sem0)
        pltpu.async_copy(ldst_hbm.at[r, s, pl.ds(blk * NR2, NR2)], idst,
                         sem1)

    def _stage_wait(blk, isrc, idst, r, sem0, sem1):
        pltpu.make_async_copy(lsrc_hbm.at[r, s, pl.ds(blk * NR2, NR2)],
                              isrc, sem0).wait()
        pltpu.make_async_copy(ldst_hbm.at[r, s, pl.ds(blk * NR2, NR2)],
                              idst, sem1).wait()

    for rpass in range(N_REL // NC):
        r = c * (N_REL // NC) + rpass

        def _zg(i, _):
            for j in range(D // 16):
                gbuf0[i, pl.ds(j * 16, 16)] = zv
            return 0
        lax.fori_loop(0, GB, _zg, 0)
        for k in range(ROWS_PT // GB):
            pltpu.async_copy(gbuf0, acc_sh.at[pl.ds(rbase + k * GB, GB)],
                             psa0)
        for k in range(ROWS_PT // GB):
            pltpu.make_async_copy(
                gbuf0, acc_sh.at[pl.ds(rbase + k * GB, GB)], psa0).wait()
        plsc.subcore_barrier()

        pltpu.sync_copy(ngrp_hbm.at[r, s], ngbuf)
        ng = ngbuf[pl.ds(0, 16)][0]
        nblk = (ng + NR2 - 1) // NR2

        @pl.when(nblk > 0)
        def _():
            _stage(0, isrc_a, idst_a, r, psa0, psa1)
            _stage_wait(0, isrc_a, idst_a, r, psa0, psa1)

        def _pstep(p, _):
            k0 = 2 * p
            k1 = k0 + 1

            @pl.when(k1 < nblk)
            def _():
                _stage(k1, isrc_b, idst_b, r, psb0, psb1)

            @pl.when(k0 >= 1)
            def _():
                _stage_wait(k0, isrc_a, idst_a, r, psa0, psa1)

            _drain_a(0, jnp.minimum(ng - k0 * NR2, NR2))

            @pl.when(k1 < nblk)
            def _():
                @pl.when(k1 + 1 < nblk)
                def _():
                    _stage(k1 + 1, isrc_a, idst_a, r, psa0, psa1)

                _stage_wait(k1, isrc_b, idst_b, r, psb0, psb1)
                _drain_b(0, jnp.minimum(ng - k1 * NR2, NR2))

            return 0

        lax.fori_loop(0, (nblk + 1) >> 1, _pstep, 0)

        plsc.subcore_barrier()
        pltpu.sync_copy(acc_sh.at[pl.ds(rbase, ROWS_PT)],
                        s_hbm.at[r, pl.ds(rbase, ROWS_PT)])
        plsc.subcore_barrier()


_sc_scatter2 = pl.kernel(
    _sc_body2,
    out_type=jax.ShapeDtypeStruct((N_REL, NP, D), jnp.float32),
    mesh=plsc.VectorSubcoreMesh(core_axis_name="c", subcore_axis_name="s"),
    compiler_params=pltpu.CompilerParams(needs_layout_passes=False),
    scratch_types=[
        pltpu.VMEM_SHARED((NP, D), jnp.float32),
        pltpu.VMEM((NR2, GB), jnp.int32),
        pltpu.VMEM((NR2, GB), jnp.int32),
        pltpu.VMEM((NR2, GB), jnp.int32),
        pltpu.VMEM((NR2, GB), jnp.int32),
        pltpu.VMEM((GB, D), jnp.float32),
        pltpu.VMEM((GB, D), jnp.float32),
        pltpu.VMEM((GB, D), jnp.float32),
        pltpu.VMEM((16,), jnp.int32),
        pltpu.SemaphoreType.DMA,
        pltpu.SemaphoreType.DMA,
        pltpu.SemaphoreType.DMA,
        pltpu.SemaphoreType.DMA,
        pltpu.SemaphoreType.DMA,
        pltpu.SemaphoreType.DMA,
        pltpu.SemaphoreType.DMA,
        pltpu.SemaphoreType.DMA,
        pltpu.SemaphoreType.DMA,
        pltpu.SemaphoreType.DMA,
    ],
)


def _tc_layer_body(s_ref, c_ref, x_ref, wrel_ref, wroot_ref, b_ref, o_ref,
                   *, relu):
    acc = jnp.dot(x_ref[...], wroot_ref[...],
                  preferred_element_type=jnp.float32) + b_ref[...]
    cnt = jnp.maximum(jnp.sum(c_ref[...], axis=1), 1.0)
    for r in range(N_REL):
        inv_r = 1.0 / cnt[r]
        mean_r = s_ref[r] * lax.broadcast_in_dim(inv_r, s_ref.shape[1:], (0,))
        acc = acc + jnp.dot(mean_r, wrel_ref[r],
                            preferred_element_type=jnp.float32)
    o_ref[...] = jnp.maximum(acc, 0.0) if relu else acc


def _tc_layer(S, C, x, W_rel, W_root, b, relu):
    k = W_rel.shape[2]
    blk = 1280
    grid = (NP // blk,)
    return pl.pallas_call(
        functools.partial(_tc_layer_body, relu=relu),
        grid=grid,
        in_specs=[
            pl.BlockSpec((N_REL, blk, D), lambda i: (0, i, 0)),
            pl.BlockSpec((N_REL, NS, blk), lambda i: (0, 0, i)),
            pl.BlockSpec((blk, D), lambda i: (i, 0)),
            pl.BlockSpec((N_REL, D, k), lambda i: (0, 0, 0)),
            pl.BlockSpec((D, k), lambda i: (0, 0)),
            pl.BlockSpec((1, k), lambda i: (0, 0)),
        ],
        out_specs=pl.BlockSpec((blk, k), lambda i: (i, 0)),
        out_shape=jax.ShapeDtypeStruct((NP, k), jnp.float32),
    )(S, C, x, W_rel, W_root, b)


def kernel(classic_features, edge_index, edge_type, W1_rel, W1_root, b1,
           W2_rel, W2_root, b2):
    src = edge_index[0].astype(jnp.int32)
    dst = edge_index[1].astype(jnp.int32)
    typ = edge_type.astype(jnp.int32)
    edges = src | (dst << 14) | (typ << 28)

    xp = jnp.zeros((NP, D), jnp.float32).at[:N_NODES].set(classic_features)
    S1, C1, LS, LD, NG = _sc_scatter(xp, edges)
    C1p = jnp.pad(C1, ((0, 0), (0, 0), (0, NP - N_NODES)))
    h = _tc_layer(S1, C1p, xp, W1_rel, W1_root, b1.reshape(1, -1), relu=True)
    S2 = _sc_scatter2(h, LS, LD, NG)
    out = _tc_layer(S2, C1p, h, W2_rel, W2_root, b2.reshape(1, -1),
                    relu=False)
    return out[:N_NODES]
